# Initial kernel scaffold; baseline (speedup 1.0000x reference)
import jax
import jax.numpy as jnp
from jax.experimental import pallas as pl


def kernel(x, edge_index, conv1_Wl, conv1_bl, conv1_Wr, conv2_Wl, conv2_bl, conv2_Wr, mlp_W1, mlp_b1, ln_g, ln_b, mlp_W2, mlp_b2):
    raise NotImplementedError("write your pallas kernel here")


# speedup vs baseline: 4.3441x; 4.3441x over previous
import functools

import jax
import jax.numpy as jnp
from jax import lax
from jax.experimental import pallas as pl
from jax.experimental.pallas import tpu as pltpu
from jax.experimental.pallas import tpu_sc as plsc

N = 10000
E = 320000
F1 = 128
FH = 128
NC = 2
NS = 16
K = 80
W = 624
ZR = 208
DEGP = 10240

_MESH = plsc.VectorSubcoreMesh(
    core_axis_name="c", subcore_axis_name="s", num_cores=NC, num_subcores=NS
)


def _fill_f32_2d(ref, rows, lanes, val):
    def body(r, carry):
        for l in range(lanes // 16):
            ref[r, pl.ds(l * 16, 16)] = jnp.full((16,), val, jnp.float32)
        return carry
    lax.fori_loop(0, rows, body, 0)


def _fill_f32_1d(ref, n, val):
    def body(i, carry):
        ref[pl.ds(i * 16, 16)] = jnp.full((16,), val, jnp.float32)
        return carry
    lax.fori_loop(0, n // 16, body, 0)


@functools.partial(
    pl.kernel,
    out_type=[
        jax.ShapeDtypeStruct((NC, N, F1), jnp.float32),
        jax.ShapeDtypeStruct((NC * N,), jnp.float32),
    ],
    mesh=_MESH,
    scratch_types=[
        pltpu.VMEM((K,), jnp.int32),
        pltpu.VMEM((K,), jnp.int32),
        pltpu.VMEM((K, F1), jnp.float32),
        pltpu.VMEM((K,), jnp.float32),
        pltpu.VMEM((ZR, F1), jnp.float32),
        pltpu.VMEM((DEGP // NS,), jnp.float32),
        pltpu.VMEM((N,), jnp.float32),
        pltpu.VMEM_SHARED((N, F1), jnp.float32),
        pltpu.VMEM_SHARED((DEGP,), jnp.float32),
        pltpu.SemaphoreType.DMA,
    ],
)
def _sc_conv1(x_hbm, src_hbm, dst_hbm, out_hbm, deg_hbm,
              idxs, idxd, rows, ones, zbuf, dzbuf, dwb, acc_sh, deg_sh, sem):
    c = lax.axis_index("c")
    s = lax.axis_index("s")

    _fill_f32_2d(zbuf, ZR, F1, 0.0)
    _fill_f32_1d(dzbuf, DEGP // NS, 0.0)
    _fill_f32_1d(ones, K, 1.0)
    for k in range(W // ZR):
        pltpu.sync_copy(zbuf, acc_sh.at[pl.ds(s * W + k * ZR, ZR)])

    @pl.when(s == NS - 1)
    def _zero_tail():
        pltpu.sync_copy(zbuf.at[pl.ds(0, N - W * NS)],
                        acc_sh.at[pl.ds(W * NS, N - W * NS)])

    pltpu.sync_copy(dzbuf, deg_sh.at[pl.ds(s * (DEGP // NS), DEGP // NS)])
    plsc.subcore_barrier()

    per_tile = E // (NC * NS)
    base = c * (E // NC) + s * per_tile
    nchunks = per_tile // K

    def chunk(i, carry):
        off = base + i * K
        pltpu.sync_copy(src_hbm.at[pl.ds(off, K)], idxs)
        pltpu.sync_copy(dst_hbm.at[pl.ds(off, K)], idxd)
        pltpu.async_copy(x_hbm.at[idxs], rows, sem).wait()
        pltpu.sync_copy(rows, acc_sh.at[idxd], add=True)
        pltpu.sync_copy(ones, deg_sh.at[idxd], add=True)
        return carry

    lax.fori_loop(0, nchunks, chunk, 0)
    plsc.subcore_barrier()

    pltpu.sync_copy(acc_sh.at[pl.ds(s * W, W)],
                    out_hbm.at[c, pl.ds(s * W, W)])

    @pl.when(s == NS - 1)
    def _writeback_tail():
        pltpu.sync_copy(acc_sh.at[pl.ds(W * NS, N - W * NS)],
                        out_hbm.at[c, pl.ds(W * NS, N - W * NS)])

    @pl.when(s == 0)
    def _writeback_deg():
        pltpu.sync_copy(deg_sh.at[pl.ds(0, N)], dwb)
        pltpu.sync_copy(dwb, deg_hbm.at[pl.ds(c * N, N)])


@functools.partial(
    pl.kernel,
    out_type=jax.ShapeDtypeStruct((NC, N, FH), jnp.float32),
    mesh=_MESH,
    scratch_types=[
        pltpu.VMEM((K,), jnp.int32),
        pltpu.VMEM((K,), jnp.int32),
        pltpu.VMEM((K, FH), jnp.float32),
        pltpu.VMEM((ZR, FH), jnp.float32),
        pltpu.VMEM_SHARED((N, FH), jnp.float32),
        pltpu.SemaphoreType.DMA,
    ],
)
def _sc_conv2(h_hbm, src_hbm, dst_hbm, out_hbm,
              idxs, idxd, rows, zbuf, acc_sh, sem):
    c = lax.axis_index("c")
    s = lax.axis_index("s")

    _fill_f32_2d(zbuf, ZR, FH, 0.0)
    for k in range(W // ZR):
        pltpu.sync_copy(zbuf, acc_sh.at[pl.ds(s * W + k * ZR, ZR)])

    @pl.when(s == NS - 1)
    def _zero_tail():
        pltpu.sync_copy(zbuf.at[pl.ds(0, N - W * NS)],
                        acc_sh.at[pl.ds(W * NS, N - W * NS)])

    plsc.subcore_barrier()

    per_tile = E // NS
    base = s * per_tile
    nchunks = per_tile // K
    row_off = c * N

    def chunk(i, carry):
        off = base + i * K
        pltpu.sync_copy(src_hbm.at[pl.ds(off, K)], idxs)
        pltpu.sync_copy(dst_hbm.at[pl.ds(off, K)], idxd)

        def add_off(j, cy):
            sl = pl.ds(j * 16, 16)
            idxs[sl] = idxs[sl] + row_off
            return cy
        lax.fori_loop(0, K // 16, add_off, 0)

        pltpu.async_copy(h_hbm.at[idxs], rows, sem).wait()
        pltpu.sync_copy(rows, acc_sh.at[idxd], add=True)
        return carry

    lax.fori_loop(0, nchunks, chunk, 0)
    plsc.subcore_barrier()

    pltpu.sync_copy(acc_sh.at[pl.ds(s * W, W)],
                    out_hbm.at[c, pl.ds(s * W, W)])

    @pl.when(s == NS - 1)
    def _writeback_tail():
        pltpu.sync_copy(acc_sh.at[pl.ds(W * NS, N - W * NS)],
                        out_hbm.at[c, pl.ds(W * NS, N - W * NS)])


_R = 2000


def _tc1_body(x_ref, agg_ref, deg_ref, wl_ref, bl_ref, wr_ref,
              h1_ref, degm_ref):
    a = agg_ref[0] + agg_ref[1]
    d = jnp.maximum(deg_ref[0] + deg_ref[1], 1.0)
    mean = a / d
    x1 = (jnp.dot(mean, wl_ref[...], preferred_element_type=jnp.float32)
          + bl_ref[...][None, :]
          + jnp.dot(x_ref[...], wr_ref[...], preferred_element_type=jnp.float32))
    h1 = jnp.maximum(x1, 0.0)
    h1_ref[0] = h1[:, :FH]
    h1_ref[1] = h1[:, FH:]
    degm_ref[...] = d


def _tc1(x, aggpair, degpair, wl, bl, wr):
    grid = (N // _R,)
    return pl.pallas_call(
        _tc1_body,
        grid=grid,
        in_specs=[
            pl.BlockSpec((_R, F1), lambda i: (i, 0)),
            pl.BlockSpec((NC, _R, F1), lambda i: (0, i, 0)),
            pl.BlockSpec((NC, _R, 1), lambda i: (0, i, 0)),
            pl.BlockSpec((F1, 2 * FH), lambda i: (0, 0)),
            pl.BlockSpec((2 * FH,), lambda i: (0,)),
            pl.BlockSpec((F1, 2 * FH), lambda i: (0, 0)),
        ],
        out_specs=[
            pl.BlockSpec((NC, _R, FH), lambda i: (0, i, 0)),
            pl.BlockSpec((_R, 1), lambda i: (i, 0)),
        ],
        out_shape=[
            jax.ShapeDtypeStruct((NC, N, FH), jnp.float32),
            jax.ShapeDtypeStruct((N, 1), jnp.float32),
        ],
    )(x, aggpair, degpair, wl, bl, wr)


def _tc2_body(x_ref, h1_ref, agg2_ref, degm_ref, wl2_ref, bl2_ref, wr2_ref,
              w1_ref, b1_ref, g_ref, b_ref, w2_ref, b2_ref, out_ref):
    d = degm_ref[...]
    h1 = jnp.concatenate([h1_ref[0], h1_ref[1]], axis=1)
    m2 = jnp.concatenate([agg2_ref[0], agg2_ref[1]], axis=1) / d
    x2 = (jnp.dot(m2, wl2_ref[...], preferred_element_type=jnp.float32)
          + bl2_ref[...][None, :]
          + jnp.dot(h1, wr2_ref[...], preferred_element_type=jnp.float32))
    h2 = jnp.maximum(x2, 0.0)
    h = jnp.concatenate([x_ref[...], h1, h2], axis=1)
    m1 = jnp.dot(h, w1_ref[...], preferred_element_type=jnp.float32)
    m1 = jnp.maximum(m1 + b1_ref[...][None, :], 0.0)
    mu = jnp.mean(m1, axis=1, keepdims=True)
    var = jnp.mean((m1 - mu) * (m1 - mu), axis=1, keepdims=True)
    ln = (m1 - mu) * lax.rsqrt(var + 1e-5) * g_ref[...][None, :] + b_ref[...][None, :]
    out_ref[...] = (jnp.dot(ln, w2_ref[...], preferred_element_type=jnp.float32)
                    + b2_ref[...][None, :])


def _tc2(x, h1pair, agg2pair, degm, wl2, bl2, wr2, w1, b1, g, b, w2, b2):
    grid = (N // _R,)
    return pl.pallas_call(
        _tc2_body,
        grid=grid,
        in_specs=[
            pl.BlockSpec((_R, F1), lambda i: (i, 0)),
            pl.BlockSpec((NC, _R, FH), lambda i: (0, i, 0)),
            pl.BlockSpec((NC, _R, FH), lambda i: (0, i, 0)),
            pl.BlockSpec((_R, 1), lambda i: (i, 0)),
            pl.BlockSpec((2 * FH, 2 * FH), lambda i: (0, 0)),
            pl.BlockSpec((2 * FH,), lambda i: (0,)),
            pl.BlockSpec((2 * FH, 2 * FH), lambda i: (0, 0)),
            pl.BlockSpec((F1 + 4 * FH, 2 * FH), lambda i: (0, 0)),
            pl.BlockSpec((2 * FH,), lambda i: (0,)),
            pl.BlockSpec((2 * FH,), lambda i: (0,)),
            pl.BlockSpec((2 * FH,), lambda i: (0,)),
            pl.BlockSpec((2 * FH, 2), lambda i: (0, 0)),
            pl.BlockSpec((2,), lambda i: (0,)),
        ],
        out_specs=pl.BlockSpec((_R, 2), lambda i: (i, 0)),
        out_shape=jax.ShapeDtypeStruct((N, 2), jnp.float32),
    )(x, h1pair, agg2pair, degm, wl2, bl2, wr2, w1, b1, g, b, w2, b2)


def kernel(x, edge_index, conv1_Wl, conv1_bl, conv1_Wr, conv2_Wl, conv2_bl,
           conv2_Wr, mlp_W1, mlp_b1, ln_g, ln_b, mlp_W2, mlp_b2):
    ei = edge_index.astype(jnp.int32)
    src = ei[0]
    dst = ei[1]

    aggpair, degpair = _sc_conv1(x, src, dst)
    h1pair, degm = _tc1(x, aggpair, degpair.reshape(NC, N, 1),
                        conv1_Wl, conv1_bl, conv1_Wr)
    agg2pair = _sc_conv2(h1pair.reshape(NC * N, FH), src, dst)
    out = _tc2(x, h1pair, agg2pair, degm, conv2_Wl, conv2_bl, conv2_Wr,
               mlp_W1, mlp_b1, ln_g, ln_b, mlp_W2, mlp_b2)
    return out


# speedup vs baseline: 10.4208x; 2.3988x over previous
import functools

import jax
import jax.numpy as jnp
from jax import lax
from jax.experimental import pallas as pl
from jax.experimental.pallas import tpu as pltpu
from jax.experimental.pallas import tpu_sc as plsc

N = 10000
E = 320000
F1 = 128
FH = 128
NC = 2
NS = 16
K = 80
W = 624
ZR = 48
DEGP = 10240

_MESH = plsc.VectorSubcoreMesh(
    core_axis_name="c", subcore_axis_name="s", num_cores=NC, num_subcores=NS
)


def _fill_f32_2d(ref, rows, lanes, val):
    def body(r, carry):
        for l in range(lanes // 16):
            ref[r, pl.ds(l * 16, 16)] = jnp.full((16,), val, jnp.float32)
        return carry
    lax.fori_loop(0, rows, body, 0)


def _fill_f32_1d(ref, n, val):
    def body(i, carry):
        ref[pl.ds(i * 16, 16)] = jnp.full((16,), val, jnp.float32)
        return carry
    lax.fori_loop(0, n // 16, body, 0)


def _agg_pipeline(table_hbm, src_hbm, dst_hbm, acc_sh, idxs, idxd, rows,
                  sem_i, sem_g, base, nchunks, row_off=None,
                  deg_sh=None, ones=None):
    def issue_loads(i, slot):
        off = base + i * K
        pltpu.async_copy(src_hbm.at[pl.ds(off, K)], idxs[slot], sem_i[slot])
        pltpu.async_copy(dst_hbm.at[pl.ds(off, K)], idxd[slot], sem_i[slot])

    def wait_loads(slot):
        pltpu.make_async_copy(src_hbm.at[pl.ds(0, K)], idxs[slot],
                              sem_i[slot]).wait()
        pltpu.make_async_copy(dst_hbm.at[pl.ds(0, K)], idxd[slot],
                              sem_i[slot]).wait()

    def start_gather(slot, rb):
        if row_off is not None:
            def add_off(j, cy):
                sl = pl.ds(j * 16, 16)
                idxs[slot][sl] = idxs[slot][sl] + row_off
                return cy
            lax.fori_loop(0, K // 16, add_off, 0)
        pltpu.async_copy(table_hbm.at[idxs[slot]], rows[rb], sem_g[rb])

    def wait_gather(rb):
        pltpu.make_async_copy(table_hbm.at[pl.ds(0, K)], rows[rb],
                              sem_g[rb]).wait()

    issue_loads(0, 0)
    issue_loads(1, 1)
    wait_loads(0)
    start_gather(0, 0)

    nquads = (nchunks + 3) // 4

    def quad(q, carry):
        for b in range(4):
            i = q * 4 + b
            s_next = (b + 1) % 4
            s_pref = (b + 2) % 4

            @pl.when(i + 1 < nchunks)
            def _gather_next():
                wait_loads(s_next)
                start_gather(s_next, (b + 1) % 2)

            @pl.when(i + 2 < nchunks)
            def _prefetch_idx():
                issue_loads(i + 2, s_pref)

            @pl.when(i < nchunks)
            def _scatter():
                wait_gather(b % 2)
                pltpu.sync_copy(rows[b % 2], acc_sh.at[idxd[b]], add=True)
                if deg_sh is not None:
                    pltpu.sync_copy(ones, deg_sh.at[idxd[b]], add=True)
        return carry

    lax.fori_loop(0, nquads, quad, 0)


@functools.partial(
    pl.kernel,
    out_type=[
        jax.ShapeDtypeStruct((NC, N, F1), jnp.float32),
        jax.ShapeDtypeStruct((NC * N,), jnp.float32),
    ],
    mesh=_MESH,
    scratch_types=[
        [pltpu.VMEM((K,), jnp.int32)] * 4,
        [pltpu.VMEM((K,), jnp.int32)] * 4,
        [pltpu.VMEM((K, F1), jnp.float32)] * 2,
        pltpu.VMEM((K,), jnp.float32),
        pltpu.VMEM((ZR, F1), jnp.float32),
        pltpu.VMEM((DEGP // NS,), jnp.float32),
        pltpu.VMEM_SHARED((N, F1), jnp.float32),
        pltpu.VMEM_SHARED((DEGP,), jnp.float32),
        [pltpu.SemaphoreType.DMA] * 4,
        [pltpu.SemaphoreType.DMA] * 2,
    ],
)
def _sc_conv1(x_hbm, src_hbm, dst_hbm, out_hbm, deg_hbm,
              idxs, idxd, rows, ones, zbuf, dzbuf, acc_sh, deg_sh,
              sem_i, sem_g):
    c = lax.axis_index("c")
    s = lax.axis_index("s")

    _fill_f32_2d(zbuf, ZR, F1, 0.0)
    _fill_f32_1d(dzbuf, DEGP // NS, 0.0)
    _fill_f32_1d(ones, K, 1.0)
    for k in range(W // ZR):
        pltpu.sync_copy(zbuf, acc_sh.at[pl.ds(s * W + k * ZR, ZR)])

    @pl.when(s == NS - 1)
    def _zero_tail():
        pltpu.sync_copy(zbuf.at[pl.ds(0, N - W * NS)],
                        acc_sh.at[pl.ds(W * NS, N - W * NS)])

    pltpu.sync_copy(dzbuf, deg_sh.at[pl.ds(s * (DEGP // NS), DEGP // NS)])
    plsc.subcore_barrier()

    per_tile = E // (NC * NS)
    base = c * (E // NC) + s * per_tile
    nchunks = per_tile // K

    _agg_pipeline(x_hbm, src_hbm, dst_hbm, acc_sh, idxs, idxd, rows,
                  sem_i, sem_g, base, nchunks, deg_sh=deg_sh, ones=ones)
    plsc.subcore_barrier()

    pltpu.sync_copy(acc_sh.at[pl.ds(s * W, W)],
                    out_hbm.at[c, pl.ds(s * W, W)])

    @pl.when(s == NS - 1)
    def _writeback_tail():
        pltpu.sync_copy(acc_sh.at[pl.ds(W * NS, N - W * NS)],
                        out_hbm.at[c, pl.ds(W * NS, N - W * NS)])

    DW = DEGP // NS

    @pl.when(s < NS - 1)
    def _writeback_deg():
        pltpu.sync_copy(deg_sh.at[pl.ds(s * DW, DW)], dzbuf)
        pltpu.sync_copy(dzbuf, deg_hbm.at[pl.ds(c * N + s * DW, DW)])

    @pl.when(s == NS - 1)
    def _writeback_deg_tail():
        pltpu.sync_copy(deg_sh.at[pl.ds((NS - 1) * DW, N - (NS - 1) * DW)],
                        dzbuf.at[pl.ds(0, N - (NS - 1) * DW)])
        pltpu.sync_copy(dzbuf.at[pl.ds(0, N - (NS - 1) * DW)],
                        deg_hbm.at[pl.ds(c * N + (NS - 1) * DW,
                                         N - (NS - 1) * DW)])


@functools.partial(
    pl.kernel,
    out_type=jax.ShapeDtypeStruct((NC, N, FH), jnp.float32),
    mesh=_MESH,
    scratch_types=[
        [pltpu.VMEM((K,), jnp.int32)] * 4,
        [pltpu.VMEM((K,), jnp.int32)] * 4,
        [pltpu.VMEM((K, FH), jnp.float32)] * 2,
        pltpu.VMEM((ZR, FH), jnp.float32),
        pltpu.VMEM_SHARED((N, FH), jnp.float32),
        [pltpu.SemaphoreType.DMA] * 4,
        [pltpu.SemaphoreType.DMA] * 2,
    ],
)
def _sc_conv2(h_hbm, src_hbm, dst_hbm, out_hbm,
              idxs, idxd, rows, zbuf, acc_sh, sem_i, sem_g):
    c = lax.axis_index("c")
    s = lax.axis_index("s")

    _fill_f32_2d(zbuf, ZR, FH, 0.0)
    for k in range(W // ZR):
        pltpu.sync_copy(zbuf, acc_sh.at[pl.ds(s * W + k * ZR, ZR)])

    @pl.when(s == NS - 1)
    def _zero_tail():
        pltpu.sync_copy(zbuf.at[pl.ds(0, N - W * NS)],
                        acc_sh.at[pl.ds(W * NS, N - W * NS)])

    plsc.subcore_barrier()

    per_tile = E // NS
    base = s * per_tile
    nchunks = per_tile // K
    row_off = c * N

    _agg_pipeline(h_hbm, src_hbm, dst_hbm, acc_sh, idxs, idxd, rows,
                  sem_i, sem_g, base, nchunks, row_off=row_off)
    plsc.subcore_barrier()

    pltpu.sync_copy(acc_sh.at[pl.ds(s * W, W)],
                    out_hbm.at[c, pl.ds(s * W, W)])

    @pl.when(s == NS - 1)
    def _writeback_tail():
        pltpu.sync_copy(acc_sh.at[pl.ds(W * NS, N - W * NS)],
                        out_hbm.at[c, pl.ds(W * NS, N - W * NS)])


_R = 2000


def _tc1_body(x_ref, agg_ref, deg_ref, wl_ref, bl_ref, wr_ref,
              h1_ref, degm_ref):
    a = agg_ref[0] + agg_ref[1]
    d = jnp.maximum(deg_ref[0] + deg_ref[1], 1.0)
    mean = a / d
    x1 = (jnp.dot(mean, wl_ref[...], preferred_element_type=jnp.float32)
          + bl_ref[...][None, :]
          + jnp.dot(x_ref[...], wr_ref[...], preferred_element_type=jnp.float32))
    h1 = jnp.maximum(x1, 0.0)
    h1_ref[0] = h1[:, :FH]
    h1_ref[1] = h1[:, FH:]
    degm_ref[...] = d


def _tc1(x, aggpair, degpair, wl, bl, wr):
    grid = (N // _R,)
    return pl.pallas_call(
        _tc1_body,
        grid=grid,
        in_specs=[
            pl.BlockSpec((_R, F1), lambda i: (i, 0)),
            pl.BlockSpec((NC, _R, F1), lambda i: (0, i, 0)),
            pl.BlockSpec((NC, _R, 1), lambda i: (0, i, 0)),
            pl.BlockSpec((F1, 2 * FH), lambda i: (0, 0)),
            pl.BlockSpec((2 * FH,), lambda i: (0,)),
            pl.BlockSpec((F1, 2 * FH), lambda i: (0, 0)),
        ],
        out_specs=[
            pl.BlockSpec((NC, _R, FH), lambda i: (0, i, 0)),
            pl.BlockSpec((_R, 1), lambda i: (i, 0)),
        ],
        out_shape=[
            jax.ShapeDtypeStruct((NC, N, FH), jnp.float32),
            jax.ShapeDtypeStruct((N, 1), jnp.float32),
        ],
    )(x, aggpair, degpair, wl, bl, wr)


def _tc2_body(x_ref, h1_ref, agg2_ref, degm_ref, wl2_ref, bl2_ref, wr2_ref,
              w1_ref, b1_ref, g_ref, b_ref, w2_ref, b2_ref, out_ref):
    d = degm_ref[...]
    h1 = jnp.concatenate([h1_ref[0], h1_ref[1]], axis=1)
    m2 = jnp.concatenate([agg2_ref[0], agg2_ref[1]], axis=1) / d
    x2 = (jnp.dot(m2, wl2_ref[...], preferred_element_type=jnp.float32)
          + bl2_ref[...][None, :]
          + jnp.dot(h1, wr2_ref[...], preferred_element_type=jnp.float32))
    h2 = jnp.maximum(x2, 0.0)
    h = jnp.concatenate([x_ref[...], h1, h2], axis=1)
    m1 = jnp.dot(h, w1_ref[...], preferred_element_type=jnp.float32)
    m1 = jnp.maximum(m1 + b1_ref[...][None, :], 0.0)
    mu = jnp.mean(m1, axis=1, keepdims=True)
    var = jnp.mean((m1 - mu) * (m1 - mu), axis=1, keepdims=True)
    ln = (m1 - mu) * lax.rsqrt(var + 1e-5) * g_ref[...][None, :] + b_ref[...][None, :]
    out_ref[...] = (jnp.dot(ln, w2_ref[...], preferred_element_type=jnp.float32)
                    + b2_ref[...][None, :])


def _tc2(x, h1pair, agg2pair, degm, wl2, bl2, wr2, w1, b1, g, b, w2, b2):
    grid = (N // _R,)
    return pl.pallas_call(
        _tc2_body,
        grid=grid,
        in_specs=[
            pl.BlockSpec((_R, F1), lambda i: (i, 0)),
            pl.BlockSpec((NC, _R, FH), lambda i: (0, i, 0)),
            pl.BlockSpec((NC, _R, FH), lambda i: (0, i, 0)),
            pl.BlockSpec((_R, 1), lambda i: (i, 0)),
            pl.BlockSpec((2 * FH, 2 * FH), lambda i: (0, 0)),
            pl.BlockSpec((2 * FH,), lambda i: (0,)),
            pl.BlockSpec((2 * FH, 2 * FH), lambda i: (0, 0)),
            pl.BlockSpec((F1 + 4 * FH, 2 * FH), lambda i: (0, 0)),
            pl.BlockSpec((2 * FH,), lambda i: (0,)),
            pl.BlockSpec((2 * FH,), lambda i: (0,)),
            pl.BlockSpec((2 * FH,), lambda i: (0,)),
            pl.BlockSpec((2 * FH, 2), lambda i: (0, 0)),
            pl.BlockSpec((2,), lambda i: (0,)),
        ],
        out_specs=pl.BlockSpec((_R, 2), lambda i: (i, 0)),
        out_shape=jax.ShapeDtypeStruct((N, 2), jnp.float32),
    )(x, h1pair, agg2pair, degm, wl2, bl2, wr2, w1, b1, g, b, w2, b2)


def kernel(x, edge_index, conv1_Wl, conv1_bl, conv1_Wr, conv2_Wl, conv2_bl,
           conv2_Wr, mlp_W1, mlp_b1, ln_g, ln_b, mlp_W2, mlp_b2):
    ei = edge_index.astype(jnp.int32)
    src = ei[0]
    dst = ei[1]

    aggpair, degpair = _sc_conv1(x, src, dst)
    h1pair, degm = _tc1(x, aggpair, degpair.reshape(NC, N, 1),
                        conv1_Wl, conv1_bl, conv1_Wr)
    agg2pair = _sc_conv2(h1pair.reshape(NC * N, FH), src, dst)
    out = _tc2(x, h1pair, agg2pair, degm, conv2_Wl, conv2_bl, conv2_Wr,
               mlp_W1, mlp_b1, ln_g, ln_b, mlp_W2, mlp_b2)
    return out


# speedup vs baseline: 10.5425x; 1.0117x over previous
import functools

import jax
import jax.numpy as jnp
from jax import lax
from jax.experimental import pallas as pl
from jax.experimental.pallas import tpu as pltpu
from jax.experimental.pallas import tpu_sc as plsc

N = 10000
E = 320000
F1 = 128
FH = 128
NC = 2
NS = 16
K = 80
W = 624
ZR = 48
DEGP = 10240

_MESH = plsc.VectorSubcoreMesh(
    core_axis_name="c", subcore_axis_name="s", num_cores=NC, num_subcores=NS
)


def _fill_f32_2d(ref, rows, lanes, val):
    def body(r, carry):
        for l in range(lanes // 16):
            ref[r, pl.ds(l * 16, 16)] = jnp.full((16,), val, jnp.float32)
        return carry
    lax.fori_loop(0, rows, body, 0)


def _fill_f32_1d(ref, n, val):
    def body(i, carry):
        ref[pl.ds(i * 16, 16)] = jnp.full((16,), val, jnp.float32)
        return carry
    lax.fori_loop(0, n // 16, body, 0)


def _agg_pipeline(table_hbm, src_hbm, dst_hbm, acc_sh, idxs, idxd, rows,
                  sem_i, sem_g, sem_s, sem_d, base, nchunks, row_off=None,
                  deg_sh=None, ones=None):
    def issue_loads(i, slot):
        off = base + i * K
        pltpu.async_copy(src_hbm.at[pl.ds(off, K)], idxs[slot], sem_i[slot])
        pltpu.async_copy(dst_hbm.at[pl.ds(off, K)], idxd[slot], sem_i[slot])

    def wait_loads(slot):
        pltpu.make_async_copy(src_hbm.at[pl.ds(0, K)], idxs[slot],
                              sem_i[slot]).wait()
        pltpu.make_async_copy(dst_hbm.at[pl.ds(0, K)], idxd[slot],
                              sem_i[slot]).wait()

    def start_gather(slot, rb):
        if row_off is not None:
            def add_off(j, cy):
                sl = pl.ds(j * 16, 16)
                idxs[slot][sl] = idxs[slot][sl] + row_off
                return cy
            lax.fori_loop(0, K // 16, add_off, 0)
        pltpu.async_copy(table_hbm.at[idxs[slot]], rows[rb], sem_g[rb])

    def wait_gather(rb):
        pltpu.make_async_copy(table_hbm.at[pl.ds(0, K)], rows[rb],
                              sem_g[rb]).wait()

    def start_scatter(slot, rb):
        pltpu.async_copy(rows[rb], acc_sh.at[idxd[slot]], sem_s[rb], add=True)
        if deg_sh is not None:
            pltpu.async_copy(ones, deg_sh.at[idxd[slot]], sem_d[rb], add=True)

    def wait_scatter(rb):
        pltpu.make_async_copy(rows[rb], acc_sh.at[pl.ds(0, K)],
                              sem_s[rb]).wait()
        if deg_sh is not None:
            pltpu.make_async_copy(ones, deg_sh.at[pl.ds(0, K)],
                                  sem_d[rb]).wait()

    issue_loads(0, 0)
    issue_loads(1, 1)
    wait_loads(0)
    start_gather(0, 0)

    nquads = (nchunks + 3) // 4

    def quad(q, carry):
        for b in range(4):
            i = q * 4 + b
            s_next = (b + 1) % 4
            s_pref = (b + 2) % 4

            @pl.when(i + 1 < nchunks)
            def _gather_next():
                wait_loads(s_next)

                @pl.when(i >= 1)
                def _drain_prev_scatter():
                    wait_scatter((b + 1) % 2)

                start_gather(s_next, (b + 1) % 2)

            @pl.when(i + 2 < nchunks)
            def _prefetch_idx():
                issue_loads(i + 2, s_pref)

            @pl.when(i < nchunks)
            def _scatter():
                wait_gather(b % 2)
                start_scatter(b, b % 2)
        return carry

    lax.fori_loop(0, nquads, quad, 0)

    wait_scatter((nchunks - 2) % 2)
    wait_scatter((nchunks - 1) % 2)


@functools.partial(
    pl.kernel,
    out_type=[
        jax.ShapeDtypeStruct((NC, N, F1), jnp.float32),
        jax.ShapeDtypeStruct((NC * N,), jnp.float32),
    ],
    mesh=_MESH,
    scratch_types=[
        [pltpu.VMEM((K,), jnp.int32)] * 4,
        [pltpu.VMEM((K,), jnp.int32)] * 4,
        [pltpu.VMEM((K, F1), jnp.float32)] * 2,
        pltpu.VMEM((K,), jnp.float32),
        pltpu.VMEM((ZR, F1), jnp.float32),
        pltpu.VMEM((DEGP // NS,), jnp.float32),
        pltpu.VMEM_SHARED((N, F1), jnp.float32),
        pltpu.VMEM_SHARED((DEGP,), jnp.float32),
        [pltpu.SemaphoreType.DMA] * 4,
        [pltpu.SemaphoreType.DMA] * 2,
        [pltpu.SemaphoreType.DMA] * 2,
        [pltpu.SemaphoreType.DMA] * 2,
    ],
)
def _sc_conv1(x_hbm, src_hbm, dst_hbm, out_hbm, deg_hbm,
              idxs, idxd, rows, ones, zbuf, dzbuf, acc_sh, deg_sh,
              sem_i, sem_g, sem_s, sem_d):
    c = lax.axis_index("c")
    s = lax.axis_index("s")

    _fill_f32_2d(zbuf, ZR, F1, 0.0)
    _fill_f32_1d(dzbuf, DEGP // NS, 0.0)
    _fill_f32_1d(ones, K, 1.0)
    for k in range(W // ZR):
        pltpu.sync_copy(zbuf, acc_sh.at[pl.ds(s * W + k * ZR, ZR)])

    @pl.when(s == NS - 1)
    def _zero_tail():
        pltpu.sync_copy(zbuf.at[pl.ds(0, N - W * NS)],
                        acc_sh.at[pl.ds(W * NS, N - W * NS)])

    pltpu.sync_copy(dzbuf, deg_sh.at[pl.ds(s * (DEGP // NS), DEGP // NS)])
    plsc.subcore_barrier()

    per_tile = E // (NC * NS)
    base = c * (E // NC) + s * per_tile
    nchunks = per_tile // K

    _agg_pipeline(x_hbm, src_hbm, dst_hbm, acc_sh, idxs, idxd, rows,
                  sem_i, sem_g, sem_s, sem_d, base, nchunks,
                  deg_sh=deg_sh, ones=ones)
    plsc.subcore_barrier()

    pltpu.sync_copy(acc_sh.at[pl.ds(s * W, W)],
                    out_hbm.at[c, pl.ds(s * W, W)])

    @pl.when(s == NS - 1)
    def _writeback_tail():
        pltpu.sync_copy(acc_sh.at[pl.ds(W * NS, N - W * NS)],
                        out_hbm.at[c, pl.ds(W * NS, N - W * NS)])

    DW = DEGP // NS

    @pl.when(s < NS - 1)
    def _writeback_deg():
        pltpu.sync_copy(deg_sh.at[pl.ds(s * DW, DW)], dzbuf)
        pltpu.sync_copy(dzbuf, deg_hbm.at[pl.ds(c * N + s * DW, DW)])

    @pl.when(s == NS - 1)
    def _writeback_deg_tail():
        pltpu.sync_copy(deg_sh.at[pl.ds((NS - 1) * DW, N - (NS - 1) * DW)],
                        dzbuf.at[pl.ds(0, N - (NS - 1) * DW)])
        pltpu.sync_copy(dzbuf.at[pl.ds(0, N - (NS - 1) * DW)],
                        deg_hbm.at[pl.ds(c * N + (NS - 1) * DW,
                                         N - (NS - 1) * DW)])


@functools.partial(
    pl.kernel,
    out_type=jax.ShapeDtypeStruct((NC, N, FH), jnp.float32),
    mesh=_MESH,
    scratch_types=[
        [pltpu.VMEM((K,), jnp.int32)] * 4,
        [pltpu.VMEM((K,), jnp.int32)] * 4,
        [pltpu.VMEM((K, FH), jnp.float32)] * 2,
        pltpu.VMEM((ZR, FH), jnp.float32),
        pltpu.VMEM_SHARED((N, FH), jnp.float32),
        [pltpu.SemaphoreType.DMA] * 4,
        [pltpu.SemaphoreType.DMA] * 2,
        [pltpu.SemaphoreType.DMA] * 2,
    ],
)
def _sc_conv2(h_hbm, src_hbm, dst_hbm, out_hbm,
              idxs, idxd, rows, zbuf, acc_sh, sem_i, sem_g, sem_s):
    c = lax.axis_index("c")
    s = lax.axis_index("s")

    _fill_f32_2d(zbuf, ZR, FH, 0.0)
    for k in range(W // ZR):
        pltpu.sync_copy(zbuf, acc_sh.at[pl.ds(s * W + k * ZR, ZR)])

    @pl.when(s == NS - 1)
    def _zero_tail():
        pltpu.sync_copy(zbuf.at[pl.ds(0, N - W * NS)],
                        acc_sh.at[pl.ds(W * NS, N - W * NS)])

    plsc.subcore_barrier()

    per_tile = E // NS
    base = s * per_tile
    nchunks = per_tile // K
    row_off = c * N

    _agg_pipeline(h_hbm, src_hbm, dst_hbm, acc_sh, idxs, idxd, rows,
                  sem_i, sem_g, sem_s, None, base, nchunks, row_off=row_off)
    plsc.subcore_barrier()

    pltpu.sync_copy(acc_sh.at[pl.ds(s * W, W)],
                    out_hbm.at[c, pl.ds(s * W, W)])

    @pl.when(s == NS - 1)
    def _writeback_tail():
        pltpu.sync_copy(acc_sh.at[pl.ds(W * NS, N - W * NS)],
                        out_hbm.at[c, pl.ds(W * NS, N - W * NS)])


_R = 2000


def _tc1_body(x_ref, agg_ref, deg_ref, wl_ref, bl_ref, wr_ref,
              h1_ref, degm_ref):
    a = agg_ref[0] + agg_ref[1]
    d = jnp.maximum(deg_ref[0] + deg_ref[1], 1.0)
    mean = a / d
    x1 = (jnp.dot(mean, wl_ref[...], preferred_element_type=jnp.float32)
          + bl_ref[...][None, :]
          + jnp.dot(x_ref[...], wr_ref[...], preferred_element_type=jnp.float32))
    h1 = jnp.maximum(x1, 0.0)
    h1_ref[0] = h1[:, :FH]
    h1_ref[1] = h1[:, FH:]
    degm_ref[...] = d


def _tc1(x, aggpair, degpair, wl, bl, wr):
    grid = (N // _R,)
    return pl.pallas_call(
        _tc1_body,
        grid=grid,
        in_specs=[
            pl.BlockSpec((_R, F1), lambda i: (i, 0)),
            pl.BlockSpec((NC, _R, F1), lambda i: (0, i, 0)),
            pl.BlockSpec((NC, _R, 1), lambda i: (0, i, 0)),
            pl.BlockSpec((F1, 2 * FH), lambda i: (0, 0)),
            pl.BlockSpec((2 * FH,), lambda i: (0,)),
            pl.BlockSpec((F1, 2 * FH), lambda i: (0, 0)),
        ],
        out_specs=[
            pl.BlockSpec((NC, _R, FH), lambda i: (0, i, 0)),
            pl.BlockSpec((_R, 1), lambda i: (i, 0)),
        ],
        out_shape=[
            jax.ShapeDtypeStruct((NC, N, FH), jnp.float32),
            jax.ShapeDtypeStruct((N, 1), jnp.float32),
        ],
    )(x, aggpair, degpair, wl, bl, wr)


def _tc2_body(x_ref, h1_ref, agg2_ref, degm_ref, wl2_ref, bl2_ref, wr2_ref,
              w1_ref, b1_ref, g_ref, b_ref, w2_ref, b2_ref, out_ref):
    d = degm_ref[...]
    h1 = jnp.concatenate([h1_ref[0], h1_ref[1]], axis=1)
    m2 = jnp.concatenate([agg2_ref[0], agg2_ref[1]], axis=1) / d
    x2 = (jnp.dot(m2, wl2_ref[...], preferred_element_type=jnp.float32)
          + bl2_ref[...][None, :]
          + jnp.dot(h1, wr2_ref[...], preferred_element_type=jnp.float32))
    h2 = jnp.maximum(x2, 0.0)
    h = jnp.concatenate([x_ref[...], h1, h2], axis=1)
    m1 = jnp.dot(h, w1_ref[...], preferred_element_type=jnp.float32)
    m1 = jnp.maximum(m1 + b1_ref[...][None, :], 0.0)
    mu = jnp.mean(m1, axis=1, keepdims=True)
    var = jnp.mean((m1 - mu) * (m1 - mu), axis=1, keepdims=True)
    ln = (m1 - mu) * lax.rsqrt(var + 1e-5) * g_ref[...][None, :] + b_ref[...][None, :]
    out_ref[...] = (jnp.dot(ln, w2_ref[...], preferred_element_type=jnp.float32)
                    + b2_ref[...][None, :])


def _tc2(x, h1pair, agg2pair, degm, wl2, bl2, wr2, w1, b1, g, b, w2, b2):
    grid = (N // _R,)
    return pl.pallas_call(
        _tc2_body,
        grid=grid,
        in_specs=[
            pl.BlockSpec((_R, F1), lambda i: (i, 0)),
            pl.BlockSpec((NC, _R, FH), lambda i: (0, i, 0)),
            pl.BlockSpec((NC, _R, FH), lambda i: (0, i, 0)),
            pl.BlockSpec((_R, 1), lambda i: (i, 0)),
            pl.BlockSpec((2 * FH, 2 * FH), lambda i: (0, 0)),
            pl.BlockSpec((2 * FH,), lambda i: (0,)),
            pl.BlockSpec((2 * FH, 2 * FH), lambda i: (0, 0)),
            pl.BlockSpec((F1 + 4 * FH, 2 * FH), lambda i: (0, 0)),
            pl.BlockSpec((2 * FH,), lambda i: (0,)),
            pl.BlockSpec((2 * FH,), lambda i: (0,)),
            pl.BlockSpec((2 * FH,), lambda i: (0,)),
            pl.BlockSpec((2 * FH, 2), lambda i: (0, 0)),
            pl.BlockSpec((2,), lambda i: (0,)),
        ],
        out_specs=pl.BlockSpec((_R, 2), lambda i: (i, 0)),
        out_shape=jax.ShapeDtypeStruct((N, 2), jnp.float32),
    )(x, h1pair, agg2pair, degm, wl2, bl2, wr2, w1, b1, g, b, w2, b2)


def kernel(x, edge_index, conv1_Wl, conv1_bl, conv1_Wr, conv2_Wl, conv2_bl,
           conv2_Wr, mlp_W1, mlp_b1, ln_g, ln_b, mlp_W2, mlp_b2):
    ei = edge_index.astype(jnp.int32)
    src = ei[0]
    dst = ei[1]

    aggpair, degpair = _sc_conv1(x, src, dst)
    h1pair, degm = _tc1(x, aggpair, degpair.reshape(NC, N, 1),
                        conv1_Wl, conv1_bl, conv1_Wr)
    agg2pair = _sc_conv2(h1pair.reshape(NC * N, FH), src, dst)
    out = _tc2(x, h1pair, agg2pair, degm, conv2_Wl, conv2_bl, conv2_Wr,
               mlp_W1, mlp_b1, ln_g, ln_b, mlp_W2, mlp_b2)
    return out


# speedup vs baseline: 10.7933x; 1.0238x over previous
import functools

import jax
import jax.numpy as jnp
from jax import lax
from jax.experimental import pallas as pl
from jax.experimental.pallas import tpu as pltpu
from jax.experimental.pallas import tpu_sc as plsc

N = 10000
E = 320000
F1 = 128
FH = 128
NC = 2
NS = 16
K = 80
W = 624
ZR = 48
DEGP = 10240

_MESH = plsc.VectorSubcoreMesh(
    core_axis_name="c", subcore_axis_name="s", num_cores=NC, num_subcores=NS
)


def _fill_f32_2d(ref, rows, lanes, val):
    def body(r, carry):
        for l in range(lanes // 16):
            ref[r, pl.ds(l * 16, 16)] = jnp.full((16,), val, jnp.float32)
        return carry
    lax.fori_loop(0, rows, body, 0)


def _fill_f32_1d(ref, n, val):
    def body(i, carry):
        ref[pl.ds(i * 16, 16)] = jnp.full((16,), val, jnp.float32)
        return carry
    lax.fori_loop(0, n // 16, body, 0)


def _agg_pipeline(table_hbm, src_hbm, dst_hbm, acc_sh, idxs, idxd, rows,
                  sem_i, sem_g, sem_s, sem_d, base, nchunks, row_off=None,
                  deg_sh=None, ones=None):
    def issue_loads(i, slot):
        off = base + i * K
        pltpu.async_copy(src_hbm.at[pl.ds(off, K)], idxs[slot], sem_i[slot])
        pltpu.async_copy(dst_hbm.at[pl.ds(E + off, K)], idxd[slot],
                         sem_i[slot])

    def wait_loads(slot):
        pltpu.make_async_copy(src_hbm.at[pl.ds(0, K)], idxs[slot],
                              sem_i[slot]).wait()
        pltpu.make_async_copy(dst_hbm.at[pl.ds(0, K)], idxd[slot],
                              sem_i[slot]).wait()

    def start_gather(slot, rb):
        if row_off is not None:
            def add_off(j, cy):
                sl = pl.ds(j * 16, 16)
                idxs[slot][sl] = idxs[slot][sl] + row_off
                return cy
            lax.fori_loop(0, K // 16, add_off, 0)
        pltpu.async_copy(table_hbm.at[idxs[slot]], rows[rb], sem_g[rb])

    def wait_gather(rb):
        pltpu.make_async_copy(table_hbm.at[pl.ds(0, K)], rows[rb],
                              sem_g[rb]).wait()

    def start_scatter(slot, rb):
        pltpu.async_copy(rows[rb], acc_sh.at[idxd[slot]], sem_s[rb], add=True)
        if deg_sh is not None:
            pltpu.async_copy(ones, deg_sh.at[idxd[slot]], sem_d[rb], add=True)

    def wait_scatter(rb):
        pltpu.make_async_copy(rows[rb], acc_sh.at[pl.ds(0, K)],
                              sem_s[rb]).wait()
        if deg_sh is not None:
            pltpu.make_async_copy(ones, deg_sh.at[pl.ds(0, K)],
                                  sem_d[rb]).wait()

    issue_loads(0, 0)
    issue_loads(1, 1)
    wait_loads(0)
    start_gather(0, 0)

    nquads = (nchunks + 3) // 4

    def quad(q, carry):
        for b in range(4):
            i = q * 4 + b
            s_next = (b + 1) % 4
            s_pref = (b + 2) % 4

            @pl.when(i + 1 < nchunks)
            def _gather_next():
                wait_loads(s_next)

                @pl.when(i >= 1)
                def _drain_prev_scatter():
                    wait_scatter((b + 1) % 2)

                start_gather(s_next, (b + 1) % 2)

            @pl.when(i + 2 < nchunks)
            def _prefetch_idx():
                issue_loads(i + 2, s_pref)

            @pl.when(i < nchunks)
            def _scatter():
                wait_gather(b % 2)
                start_scatter(b, b % 2)
        return carry

    lax.fori_loop(0, nquads, quad, 0)

    wait_scatter((nchunks - 2) % 2)
    wait_scatter((nchunks - 1) % 2)


@functools.partial(
    pl.kernel,
    out_type=[
        jax.ShapeDtypeStruct((NC, N, F1), jnp.float32),
        jax.ShapeDtypeStruct((NC * N,), jnp.float32),
    ],
    mesh=_MESH,
    scratch_types=[
        [pltpu.VMEM((K,), jnp.int32)] * 4,
        [pltpu.VMEM((K,), jnp.int32)] * 4,
        [pltpu.VMEM((K, F1), jnp.float32)] * 2,
        pltpu.VMEM((K,), jnp.float32),
        pltpu.VMEM((ZR, F1), jnp.float32),
        pltpu.VMEM((DEGP // NS,), jnp.float32),
        pltpu.VMEM_SHARED((N, F1), jnp.float32),
        pltpu.VMEM_SHARED((DEGP,), jnp.float32),
        [pltpu.SemaphoreType.DMA] * 4,
        [pltpu.SemaphoreType.DMA] * 2,
        [pltpu.SemaphoreType.DMA] * 2,
        [pltpu.SemaphoreType.DMA] * 2,
    ],
)
def _sc_conv1(x_hbm, edges_hbm, out_hbm, deg_hbm,
              idxs, idxd, rows, ones, zbuf, dzbuf, acc_sh, deg_sh,
              sem_i, sem_g, sem_s, sem_d):
    c = lax.axis_index("c")
    s = lax.axis_index("s")

    _fill_f32_2d(zbuf, ZR, F1, 0.0)
    _fill_f32_1d(dzbuf, DEGP // NS, 0.0)
    _fill_f32_1d(ones, K, 1.0)
    for k in range(W // ZR):
        pltpu.sync_copy(zbuf, acc_sh.at[pl.ds(s * W + k * ZR, ZR)])

    @pl.when(s == NS - 1)
    def _zero_tail():
        pltpu.sync_copy(zbuf.at[pl.ds(0, N - W * NS)],
                        acc_sh.at[pl.ds(W * NS, N - W * NS)])

    pltpu.sync_copy(dzbuf, deg_sh.at[pl.ds(s * (DEGP // NS), DEGP // NS)])
    plsc.subcore_barrier()

    per_tile = E // (NC * NS)
    base = c * (E // NC) + s * per_tile
    nchunks = per_tile // K

    _agg_pipeline(x_hbm, edges_hbm, edges_hbm, acc_sh, idxs, idxd, rows,
                  sem_i, sem_g, sem_s, sem_d, base, nchunks,
                  deg_sh=deg_sh, ones=ones)
    plsc.subcore_barrier()

    pltpu.sync_copy(acc_sh.at[pl.ds(s * W, W)],
                    out_hbm.at[c, pl.ds(s * W, W)])

    @pl.when(s == NS - 1)
    def _writeback_tail():
        pltpu.sync_copy(acc_sh.at[pl.ds(W * NS, N - W * NS)],
                        out_hbm.at[c, pl.ds(W * NS, N - W * NS)])

    DW = DEGP // NS

    @pl.when(s < NS - 1)
    def _writeback_deg():
        pltpu.sync_copy(deg_sh.at[pl.ds(s * DW, DW)], dzbuf)
        pltpu.sync_copy(dzbuf, deg_hbm.at[pl.ds(c * N + s * DW, DW)])

    @pl.when(s == NS - 1)
    def _writeback_deg_tail():
        pltpu.sync_copy(deg_sh.at[pl.ds((NS - 1) * DW, N - (NS - 1) * DW)],
                        dzbuf.at[pl.ds(0, N - (NS - 1) * DW)])
        pltpu.sync_copy(dzbuf.at[pl.ds(0, N - (NS - 1) * DW)],
                        deg_hbm.at[pl.ds(c * N + (NS - 1) * DW,
                                         N - (NS - 1) * DW)])


@functools.partial(
    pl.kernel,
    out_type=jax.ShapeDtypeStruct((NC, N, FH), jnp.float32),
    mesh=_MESH,
    scratch_types=[
        [pltpu.VMEM((K,), jnp.int32)] * 4,
        [pltpu.VMEM((K,), jnp.int32)] * 4,
        [pltpu.VMEM((K, FH), jnp.float32)] * 2,
        pltpu.VMEM((ZR, FH), jnp.float32),
        pltpu.VMEM_SHARED((N, FH), jnp.float32),
        [pltpu.SemaphoreType.DMA] * 4,
        [pltpu.SemaphoreType.DMA] * 2,
        [pltpu.SemaphoreType.DMA] * 2,
    ],
)
def _sc_conv2(h_hbm, edges_hbm, out_hbm,
              idxs, idxd, rows, zbuf, acc_sh, sem_i, sem_g, sem_s):
    c = lax.axis_index("c")
    s = lax.axis_index("s")

    _fill_f32_2d(zbuf, ZR, FH, 0.0)
    for k in range(W // ZR):
        pltpu.sync_copy(zbuf, acc_sh.at[pl.ds(s * W + k * ZR, ZR)])

    @pl.when(s == NS - 1)
    def _zero_tail():
        pltpu.sync_copy(zbuf.at[pl.ds(0, N - W * NS)],
                        acc_sh.at[pl.ds(W * NS, N - W * NS)])

    plsc.subcore_barrier()

    per_tile = E // NS
    base = s * per_tile
    nchunks = per_tile // K
    row_off = c * N

    _agg_pipeline(h_hbm, edges_hbm, edges_hbm, acc_sh, idxs, idxd, rows,
                  sem_i, sem_g, sem_s, None, base, nchunks, row_off=row_off)
    plsc.subcore_barrier()

    pltpu.sync_copy(acc_sh.at[pl.ds(s * W, W)],
                    out_hbm.at[c, pl.ds(s * W, W)])

    @pl.when(s == NS - 1)
    def _writeback_tail():
        pltpu.sync_copy(acc_sh.at[pl.ds(W * NS, N - W * NS)],
                        out_hbm.at[c, pl.ds(W * NS, N - W * NS)])


_R = 2000


def _bdot(a, b):
    return jnp.dot(a.astype(jnp.bfloat16), b.astype(jnp.bfloat16),
                   preferred_element_type=jnp.float32)


def _tc1_body(x_ref, agg_ref, deg_ref, wl_ref, bl_ref, wr_ref,
              h1_ref, degm_ref):
    a = agg_ref[0] + agg_ref[1]
    d = jnp.maximum(deg_ref[0] + deg_ref[1], 1.0)
    mean = a / d
    x1 = (_bdot(mean, wl_ref[...])
          + bl_ref[...][None, :]
          + _bdot(x_ref[...], wr_ref[...]))
    h1 = jnp.maximum(x1, 0.0)
    h1_ref[0] = h1[:, :FH]
    h1_ref[1] = h1[:, FH:]
    degm_ref[...] = d


def _tc1(x, aggpair, degpair, wl, bl, wr):
    grid = (N // _R,)
    return pl.pallas_call(
        _tc1_body,
        grid=grid,
        in_specs=[
            pl.BlockSpec((_R, F1), lambda i: (i, 0)),
            pl.BlockSpec((NC, _R, F1), lambda i: (0, i, 0)),
            pl.BlockSpec((NC, _R, 1), lambda i: (0, i, 0)),
            pl.BlockSpec((F1, 2 * FH), lambda i: (0, 0)),
            pl.BlockSpec((2 * FH,), lambda i: (0,)),
            pl.BlockSpec((F1, 2 * FH), lambda i: (0, 0)),
        ],
        out_specs=[
            pl.BlockSpec((NC, _R, FH), lambda i: (0, i, 0)),
            pl.BlockSpec((_R, 1), lambda i: (i, 0)),
        ],
        out_shape=[
            jax.ShapeDtypeStruct((NC, N, FH), jnp.float32),
            jax.ShapeDtypeStruct((N, 1), jnp.float32),
        ],
    )(x, aggpair, degpair, wl, bl, wr)


def _tc2_body(x_ref, h1_ref, agg2_ref, degm_ref, wl2_ref, bl2_ref, wr2_ref,
              w1_ref, b1_ref, g_ref, b_ref, w2_ref, b2_ref, out_ref):
    d = degm_ref[...]
    h1 = jnp.concatenate([h1_ref[0], h1_ref[1]], axis=1)
    m2 = jnp.concatenate([agg2_ref[0], agg2_ref[1]], axis=1) / d
    x2 = (_bdot(m2, wl2_ref[...])
          + bl2_ref[...][None, :]
          + _bdot(h1, wr2_ref[...]))
    h2 = jnp.maximum(x2, 0.0)
    h = jnp.concatenate([x_ref[...], h1, h2], axis=1)
    m1 = _bdot(h, w1_ref[...])
    m1 = jnp.maximum(m1 + b1_ref[...][None, :], 0.0)
    mu = jnp.mean(m1, axis=1, keepdims=True)
    var = jnp.mean((m1 - mu) * (m1 - mu), axis=1, keepdims=True)
    ln = (m1 - mu) * lax.rsqrt(var + 1e-5) * g_ref[...][None, :] + b_ref[...][None, :]
    out_ref[...] = _bdot(ln, w2_ref[...]) + b2_ref[...][None, :]


def _tc2(x, h1pair, agg2pair, degm, wl2, bl2, wr2, w1, b1, g, b, w2, b2):
    grid = (N // _R,)
    return pl.pallas_call(
        _tc2_body,
        grid=grid,
        in_specs=[
            pl.BlockSpec((_R, F1), lambda i: (i, 0)),
            pl.BlockSpec((NC, _R, FH), lambda i: (0, i, 0)),
            pl.BlockSpec((NC, _R, FH), lambda i: (0, i, 0)),
            pl.BlockSpec((_R, 1), lambda i: (i, 0)),
            pl.BlockSpec((2 * FH, 2 * FH), lambda i: (0, 0)),
            pl.BlockSpec((2 * FH,), lambda i: (0,)),
            pl.BlockSpec((2 * FH, 2 * FH), lambda i: (0, 0)),
            pl.BlockSpec((F1 + 4 * FH, 2 * FH), lambda i: (0, 0)),
            pl.BlockSpec((2 * FH,), lambda i: (0,)),
            pl.BlockSpec((2 * FH,), lambda i: (0,)),
            pl.BlockSpec((2 * FH,), lambda i: (0,)),
            pl.BlockSpec((2 * FH, 2), lambda i: (0, 0)),
            pl.BlockSpec((2,), lambda i: (0,)),
        ],
        out_specs=pl.BlockSpec((_R, 2), lambda i: (i, 0)),
        out_shape=jax.ShapeDtypeStruct((N, 2), jnp.float32),
    )(x, h1pair, agg2pair, degm, wl2, bl2, wr2, w1, b1, g, b, w2, b2)


def kernel(x, edge_index, conv1_Wl, conv1_bl, conv1_Wr, conv2_Wl, conv2_bl,
           conv2_Wr, mlp_W1, mlp_b1, ln_g, ln_b, mlp_W2, mlp_b2):
    eflat = edge_index.astype(jnp.int32).reshape(2 * E)

    aggpair, degpair = _sc_conv1(x, eflat)
    h1pair, degm = _tc1(x, aggpair, degpair.reshape(NC, N, 1),
                        conv1_Wl, conv1_bl, conv1_Wr)
    agg2pair = _sc_conv2(h1pair.reshape(NC * N, FH), eflat)
    out = _tc2(x, h1pair, agg2pair, degm, conv2_Wl, conv2_bl, conv2_Wr,
               mlp_W1, mlp_b1, ln_g, ln_b, mlp_W2, mlp_b2)
    return out


# speedup vs baseline: 11.2413x; 1.0415x over previous
import functools

import jax
import jax.numpy as jnp
from jax import lax
from jax.experimental import pallas as pl
from jax.experimental.pallas import tpu as pltpu
from jax.experimental.pallas import tpu_sc as plsc

N = 10000
E = 320000
F1 = 128
FH = 128
NC = 2
NS = 16
K = 80
W = 624
ZR = 48
DEGP = 10240

_MESH = plsc.VectorSubcoreMesh(
    core_axis_name="c", subcore_axis_name="s", num_cores=NC, num_subcores=NS
)


def _fill_f32_2d(ref, rows, lanes, val):
    def body(r, carry):
        for l in range(lanes // 16):
            ref[r, pl.ds(l * 16, 16)] = jnp.full((16,), val, jnp.float32)
        return carry
    lax.fori_loop(0, rows, body, 0)


def _fill_f32_1d(ref, n, val):
    def body(i, carry):
        ref[pl.ds(i * 16, 16)] = jnp.full((16,), val, jnp.float32)
        return carry
    lax.fori_loop(0, n // 16, body, 0)


def _agg_pipeline(table_hbm, src_hbm, dst_hbm, acc_sh, idxs, idxd, rows,
                  sem_i, sem_g, sem_s, sem_d, base, nchunks, row_off=None,
                  deg_sh=None, ones=None):
    def issue_loads(i, slot):
        off = base + i * K
        pltpu.async_copy(src_hbm.at[pl.ds(off, K)], idxs[slot], sem_i[slot])
        pltpu.async_copy(dst_hbm.at[pl.ds(E + off, K)], idxd[slot],
                         sem_i[slot])

    def wait_loads(slot):
        pltpu.make_async_copy(src_hbm.at[pl.ds(0, K)], idxs[slot],
                              sem_i[slot]).wait()
        pltpu.make_async_copy(dst_hbm.at[pl.ds(0, K)], idxd[slot],
                              sem_i[slot]).wait()

    def start_gather(slot, rb):
        if row_off is not None:
            def add_off(j, cy):
                sl = pl.ds(j * 16, 16)
                idxs[slot][sl] = idxs[slot][sl] + row_off
                return cy
            lax.fori_loop(0, K // 16, add_off, 0)
        pltpu.async_copy(table_hbm.at[idxs[slot]], rows[rb], sem_g[rb])

    def wait_gather(rb):
        pltpu.make_async_copy(table_hbm.at[pl.ds(0, K)], rows[rb],
                              sem_g[rb]).wait()

    def start_scatter(slot, rb):
        pltpu.async_copy(rows[rb], acc_sh.at[idxd[slot]], sem_s[rb], add=True)
        if deg_sh is not None:
            pltpu.async_copy(ones, deg_sh.at[idxd[slot]], sem_d[rb], add=True)

    def wait_scatter(rb):
        pltpu.make_async_copy(rows[rb], acc_sh.at[pl.ds(0, K)],
                              sem_s[rb]).wait()
        if deg_sh is not None:
            pltpu.make_async_copy(ones, deg_sh.at[pl.ds(0, K)],
                                  sem_d[rb]).wait()

    issue_loads(0, 0)
    issue_loads(1, 1)
    wait_loads(0)
    start_gather(0, 0)

    nquads = (nchunks + 3) // 4

    def quad(q, carry):
        for b in range(4):
            i = q * 4 + b
            s_next = (b + 1) % 4
            s_pref = (b + 2) % 4

            @pl.when(i + 1 < nchunks)
            def _gather_next():
                wait_loads(s_next)

                @pl.when(i >= 1)
                def _drain_prev_scatter():
                    wait_scatter((b + 1) % 2)

                start_gather(s_next, (b + 1) % 2)

            @pl.when(i + 2 < nchunks)
            def _prefetch_idx():
                issue_loads(i + 2, s_pref)

            @pl.when(i < nchunks)
            def _scatter():
                wait_gather(b % 2)
                start_scatter(b, b % 2)
        return carry

    lax.fori_loop(0, nquads, quad, 0)

    wait_scatter((nchunks - 2) % 2)
    wait_scatter((nchunks - 1) % 2)


@functools.partial(
    pl.kernel,
    out_type=[
        jax.ShapeDtypeStruct((NC, N, F1), jnp.float32),
        jax.ShapeDtypeStruct((NC * DEGP,), jnp.float32),
    ],
    mesh=_MESH,
    scratch_types=[
        [pltpu.VMEM((K,), jnp.int32)] * 4,
        [pltpu.VMEM((K,), jnp.int32)] * 4,
        [pltpu.VMEM((K, F1), jnp.float32)] * 2,
        pltpu.VMEM((K,), jnp.float32),
        pltpu.VMEM((ZR, F1), jnp.float32),
        pltpu.VMEM((DEGP // NS,), jnp.float32),
        pltpu.VMEM_SHARED((N, F1), jnp.float32),
        pltpu.VMEM_SHARED((DEGP,), jnp.float32),
        [pltpu.SemaphoreType.DMA] * 4,
        [pltpu.SemaphoreType.DMA] * 2,
        [pltpu.SemaphoreType.DMA] * 2,
        [pltpu.SemaphoreType.DMA] * 2,
    ],
)
def _sc_conv1(x_hbm, edges_hbm, out_hbm, deg_hbm,
              idxs, idxd, rows, ones, zbuf, dzbuf, acc_sh, deg_sh,
              sem_i, sem_g, sem_s, sem_d):
    c = lax.axis_index("c")
    s = lax.axis_index("s")

    _fill_f32_2d(zbuf, ZR, F1, 0.0)
    _fill_f32_1d(dzbuf, DEGP // NS, 0.0)
    _fill_f32_1d(ones, K, 1.0)
    for k in range(W // ZR):
        pltpu.sync_copy(zbuf, acc_sh.at[pl.ds(s * W + k * ZR, ZR)])

    @pl.when(s == NS - 1)
    def _zero_tail():
        pltpu.sync_copy(zbuf.at[pl.ds(0, N - W * NS)],
                        acc_sh.at[pl.ds(W * NS, N - W * NS)])

    pltpu.sync_copy(dzbuf, deg_sh.at[pl.ds(s * (DEGP // NS), DEGP // NS)])
    plsc.subcore_barrier()

    per_tile = E // (NC * NS)
    base = c * (E // NC) + s * per_tile
    nchunks = per_tile // K

    _agg_pipeline(x_hbm, edges_hbm, edges_hbm, acc_sh, idxs, idxd, rows,
                  sem_i, sem_g, sem_s, sem_d, base, nchunks,
                  deg_sh=deg_sh, ones=ones)
    plsc.subcore_barrier()

    pltpu.sync_copy(acc_sh.at[pl.ds(s * W, W)],
                    out_hbm.at[c, pl.ds(s * W, W)])

    @pl.when(s == NS - 1)
    def _writeback_tail():
        pltpu.sync_copy(acc_sh.at[pl.ds(W * NS, N - W * NS)],
                        out_hbm.at[c, pl.ds(W * NS, N - W * NS)])

    DW = DEGP // NS
    pltpu.sync_copy(deg_sh.at[pl.ds(s * DW, DW)], dzbuf)
    pltpu.sync_copy(dzbuf, deg_hbm.at[pl.ds(c * DEGP + s * DW, DW)])


@functools.partial(
    pl.kernel,
    out_type=jax.ShapeDtypeStruct((NC, N, FH), jnp.float32),
    mesh=_MESH,
    scratch_types=[
        [pltpu.VMEM((K,), jnp.int32)] * 4,
        [pltpu.VMEM((K,), jnp.int32)] * 4,
        [pltpu.VMEM((K, FH), jnp.float32)] * 2,
        pltpu.VMEM((ZR, FH), jnp.float32),
        pltpu.VMEM_SHARED((N, FH), jnp.float32),
        [pltpu.SemaphoreType.DMA] * 4,
        [pltpu.SemaphoreType.DMA] * 2,
        [pltpu.SemaphoreType.DMA] * 2,
    ],
)
def _sc_conv2(h_hbm, edges_hbm, out_hbm,
              idxs, idxd, rows, zbuf, acc_sh, sem_i, sem_g, sem_s):
    c = lax.axis_index("c")
    s = lax.axis_index("s")

    _fill_f32_2d(zbuf, ZR, FH, 0.0)
    for k in range(W // ZR):
        pltpu.sync_copy(zbuf, acc_sh.at[pl.ds(s * W + k * ZR, ZR)])

    @pl.when(s == NS - 1)
    def _zero_tail():
        pltpu.sync_copy(zbuf.at[pl.ds(0, N - W * NS)],
                        acc_sh.at[pl.ds(W * NS, N - W * NS)])

    plsc.subcore_barrier()

    per_tile = E // NS
    base = s * per_tile
    nchunks = per_tile // K
    row_off = c * N

    _agg_pipeline(h_hbm, edges_hbm, edges_hbm, acc_sh, idxs, idxd, rows,
                  sem_i, sem_g, sem_s, None, base, nchunks, row_off=row_off)
    plsc.subcore_barrier()

    pltpu.sync_copy(acc_sh.at[pl.ds(s * W, W)],
                    out_hbm.at[c, pl.ds(s * W, W)])

    @pl.when(s == NS - 1)
    def _writeback_tail():
        pltpu.sync_copy(acc_sh.at[pl.ds(W * NS, N - W * NS)],
                        out_hbm.at[c, pl.ds(W * NS, N - W * NS)])


_R = 2048
_DR = DEGP // 128


def _bdot(a, b):
    return jnp.dot(a.astype(jnp.bfloat16), b.astype(jnp.bfloat16),
                   preferred_element_type=jnp.float32)


def _tc1_body(x_ref, agg_ref, deg_ref, wl_ref, bl_ref, wr_ref,
              h1_ref, degm_ref):
    a = agg_ref[0] + agg_ref[1]
    dm = jnp.maximum(deg_ref[0] + deg_ref[1], 1.0)
    inv = (1.0 / dm)[:, :, None]
    mean = (a.reshape(_R // 128, 128, F1) * inv).reshape(_R, F1)
    x1 = (_bdot(mean, wl_ref[...])
          + bl_ref[...][None, :]
          + _bdot(x_ref[...], wr_ref[...]))
    h1 = jnp.maximum(x1, 0.0)
    h1_ref[0] = h1[:, :FH]
    h1_ref[1] = h1[:, FH:]
    degm_ref[...] = dm


def _tc1(x, aggpair, degpair, wl, bl, wr):
    grid = (pl.cdiv(N, _R),)
    return pl.pallas_call(
        _tc1_body,
        grid=grid,
        in_specs=[
            pl.BlockSpec((_R, F1), lambda i: (i, 0)),
            pl.BlockSpec((NC, _R, F1), lambda i: (0, i, 0)),
            pl.BlockSpec((NC, _R // 128, 128), lambda i: (0, i, 0)),
            pl.BlockSpec((F1, 2 * FH), lambda i: (0, 0)),
            pl.BlockSpec((2 * FH,), lambda i: (0,)),
            pl.BlockSpec((F1, 2 * FH), lambda i: (0, 0)),
        ],
        out_specs=[
            pl.BlockSpec((NC, _R, FH), lambda i: (0, i, 0)),
            pl.BlockSpec((_R // 128, 128), lambda i: (i, 0)),
        ],
        out_shape=[
            jax.ShapeDtypeStruct((NC, N, FH), jnp.float32),
            jax.ShapeDtypeStruct((_DR, 128), jnp.float32),
        ],
    )(x, aggpair, degpair, wl, bl, wr)


def _tc2_body(x_ref, h1_ref, agg2_ref, degm_ref, wl2_ref, bl2_ref, wr2_ref,
              w1_ref, b1_ref, g_ref, b_ref, w2_ref, b2_ref, out_ref):
    inv = (1.0 / degm_ref[...])[:, :, None]
    h1 = jnp.concatenate([h1_ref[0], h1_ref[1]], axis=1)
    a2 = jnp.concatenate([agg2_ref[0], agg2_ref[1]], axis=1)
    m2 = (a2.reshape(_R // 128, 128, 2 * FH) * inv).reshape(_R, 2 * FH)
    x2 = (_bdot(m2, wl2_ref[...])
          + bl2_ref[...][None, :]
          + _bdot(h1, wr2_ref[...]))
    h2 = jnp.maximum(x2, 0.0)
    h = jnp.concatenate([x_ref[...], h1, h2], axis=1)
    m1 = _bdot(h, w1_ref[...])
    m1 = jnp.maximum(m1 + b1_ref[...][None, :], 0.0)
    mu = jnp.mean(m1, axis=1, keepdims=True)
    var = jnp.mean((m1 - mu) * (m1 - mu), axis=1, keepdims=True)
    ln = (m1 - mu) * lax.rsqrt(var + 1e-5) * g_ref[...][None, :] + b_ref[...][None, :]
    out_ref[...] = _bdot(ln, w2_ref[...]) + b2_ref[...][None, :]


def _tc2(x, h1pair, agg2pair, degm, wl2, bl2, wr2, w1, b1, g, b, w2, b2):
    grid = (pl.cdiv(N, _R),)
    return pl.pallas_call(
        _tc2_body,
        grid=grid,
        in_specs=[
            pl.BlockSpec((_R, F1), lambda i: (i, 0)),
            pl.BlockSpec((NC, _R, FH), lambda i: (0, i, 0)),
            pl.BlockSpec((NC, _R, FH), lambda i: (0, i, 0)),
            pl.BlockSpec((_R // 128, 128), lambda i: (i, 0)),
            pl.BlockSpec((2 * FH, 2 * FH), lambda i: (0, 0)),
            pl.BlockSpec((2 * FH,), lambda i: (0,)),
            pl.BlockSpec((2 * FH, 2 * FH), lambda i: (0, 0)),
            pl.BlockSpec((F1 + 4 * FH, 2 * FH), lambda i: (0, 0)),
            pl.BlockSpec((2 * FH,), lambda i: (0,)),
            pl.BlockSpec((2 * FH,), lambda i: (0,)),
            pl.BlockSpec((2 * FH,), lambda i: (0,)),
            pl.BlockSpec((2 * FH, 2), lambda i: (0, 0)),
            pl.BlockSpec((2,), lambda i: (0,)),
        ],
        out_specs=pl.BlockSpec((_R, 2), lambda i: (i, 0)),
        out_shape=jax.ShapeDtypeStruct((N, 2), jnp.float32),
    )(x, h1pair, agg2pair, degm, wl2, bl2, wr2, w1, b1, g, b, w2, b2)


def kernel(x, edge_index, conv1_Wl, conv1_bl, conv1_Wr, conv2_Wl, conv2_bl,
           conv2_Wr, mlp_W1, mlp_b1, ln_g, ln_b, mlp_W2, mlp_b2):
    eflat = edge_index.astype(jnp.int32).reshape(2 * E)

    aggpair, degflat = _sc_conv1(x, eflat)
    h1pair, degm = _tc1(x, aggpair, degflat.reshape(NC, _DR, 128),
                        conv1_Wl, conv1_bl, conv1_Wr)
    agg2pair = _sc_conv2(h1pair.reshape(NC * N, FH), eflat)
    out = _tc2(x, h1pair, agg2pair, degm, conv2_Wl, conv2_bl, conv2_Wr,
               mlp_W1, mlp_b1, ln_g, ln_b, mlp_W2, mlp_b2)
    return out


# speedup vs baseline: 12.3236x; 1.0963x over previous
import functools

import jax
import jax.numpy as jnp
from jax import lax
from jax.experimental import pallas as pl
from jax.experimental.pallas import tpu as pltpu
from jax.experimental.pallas import tpu_sc as plsc

N = 10000
E = 320000
F1 = 128
FH = 128
NC = 2
NS = 16
K = 128
W = 624
ZR = 48
DEGP = 10240

_MESH = plsc.VectorSubcoreMesh(
    core_axis_name="c", subcore_axis_name="s", num_cores=NC, num_subcores=NS
)


def _fill_f32_2d(ref, rows, lanes, val):
    def body(r, carry):
        for l in range(lanes // 16):
            ref[r, pl.ds(l * 16, 16)] = jnp.full((16,), val, jnp.float32)
        return carry
    lax.fori_loop(0, rows, body, 0)


def _fill_f32_1d(ref, n, val):
    def body(i, carry):
        ref[pl.ds(i * 16, 16)] = jnp.full((16,), val, jnp.float32)
        return carry
    lax.fori_loop(0, n // 16, body, 0)


def _agg_pipeline(table_hbm, src_hbm, dst_hbm, acc_sh, idxs, idxd, rows,
                  sem_i, sem_g, sem_s, sem_d, base, nchunks, row_off=None,
                  deg_sh=None, ones=None):
    def issue_loads(i, slot):
        off = base + i * K
        pltpu.async_copy(src_hbm.at[pl.ds(off, K)], idxs[slot], sem_i[slot])
        pltpu.async_copy(dst_hbm.at[pl.ds(E + off, K)], idxd[slot],
                         sem_i[slot])

    def wait_loads(slot):
        pltpu.make_async_copy(src_hbm.at[pl.ds(0, K)], idxs[slot],
                              sem_i[slot]).wait()
        pltpu.make_async_copy(dst_hbm.at[pl.ds(0, K)], idxd[slot],
                              sem_i[slot]).wait()

    def start_gather(slot, rb):
        if row_off is not None:
            def add_off(j, cy):
                sl = pl.ds(j * 16, 16)
                idxs[slot][sl] = idxs[slot][sl] + row_off
                return cy
            lax.fori_loop(0, K // 16, add_off, 0)
        pltpu.async_copy(table_hbm.at[idxs[slot]], rows[rb], sem_g[rb])

    def wait_gather(rb):
        pltpu.make_async_copy(table_hbm.at[pl.ds(0, K)], rows[rb],
                              sem_g[rb]).wait()

    def start_scatter(slot, rb):
        pltpu.async_copy(rows[rb], acc_sh.at[idxd[slot]], sem_s[rb], add=True)
        if deg_sh is not None:
            pltpu.async_copy(ones, deg_sh.at[idxd[slot]], sem_d[rb], add=True)

    def wait_scatter(rb):
        pltpu.make_async_copy(rows[rb], acc_sh.at[pl.ds(0, K)],
                              sem_s[rb]).wait()
        if deg_sh is not None:
            pltpu.make_async_copy(ones, deg_sh.at[pl.ds(0, K)],
                                  sem_d[rb]).wait()

    issue_loads(0, 0)
    issue_loads(1, 1)
    wait_loads(0)
    start_gather(0, 0)

    nquads = (nchunks + 3) // 4

    def quad(q, carry):
        for b in range(4):
            i = q * 4 + b
            s_next = (b + 1) % 4
            s_pref = (b + 2) % 4

            @pl.when(i + 1 < nchunks)
            def _gather_next():
                wait_loads(s_next)

                @pl.when(i >= 1)
                def _drain_prev_scatter():
                    wait_scatter((b + 1) % 2)

                start_gather(s_next, (b + 1) % 2)

            @pl.when(i + 2 < nchunks)
            def _prefetch_idx():
                issue_loads(i + 2, s_pref)

            @pl.when(i < nchunks)
            def _scatter():
                wait_gather(b % 2)
                start_scatter(b, b % 2)
        return carry

    lax.fori_loop(0, nquads, quad, 0)

    wait_scatter((nchunks - 2) % 2)
    wait_scatter((nchunks - 1) % 2)


def _agg_tail(table_hbm, edges_hbm, acc_sh, idxs_t, idxd_t, rows, sem_g,
              base, kt, row_off=None, deg_sh=None, ones=None):
    pltpu.sync_copy(edges_hbm.at[pl.ds(base, kt)], idxs_t)
    pltpu.sync_copy(edges_hbm.at[pl.ds(E + base, kt)], idxd_t)
    if row_off is not None:
        def add_off(j, cy):
            sl = pl.ds(j * 16, 16)
            idxs_t[sl] = idxs_t[sl] + row_off
            return cy
        lax.fori_loop(0, kt // 16, add_off, 0)
    pltpu.async_copy(table_hbm.at[idxs_t], rows[0].at[pl.ds(0, kt)],
                     sem_g[0]).wait()
    pltpu.sync_copy(rows[0].at[pl.ds(0, kt)], acc_sh.at[idxd_t], add=True)
    if deg_sh is not None:
        pltpu.sync_copy(ones.at[pl.ds(0, kt)], deg_sh.at[idxd_t], add=True)


@functools.partial(
    pl.kernel,
    out_type=[
        jax.ShapeDtypeStruct((NC, N, F1), jnp.float32),
        jax.ShapeDtypeStruct((NC * DEGP,), jnp.float32),
    ],
    mesh=_MESH,
    scratch_types=[
        [pltpu.VMEM((K,), jnp.int32)] * 4,
        [pltpu.VMEM((K,), jnp.int32)] * 4,
        [pltpu.VMEM((K, F1), jnp.float32)] * 2,
        pltpu.VMEM((K,), jnp.float32),
        pltpu.VMEM((ZR, F1), jnp.float32),
        pltpu.VMEM((DEGP // NS,), jnp.float32),
        [pltpu.VMEM((16,), jnp.int32)] * 2,
        pltpu.VMEM_SHARED((N, F1), jnp.float32),
        pltpu.VMEM_SHARED((DEGP,), jnp.float32),
        [pltpu.SemaphoreType.DMA] * 4,
        [pltpu.SemaphoreType.DMA] * 2,
        [pltpu.SemaphoreType.DMA] * 2,
        [pltpu.SemaphoreType.DMA] * 2,
    ],
)
def _sc_conv1(x_hbm, edges_hbm, out_hbm, deg_hbm,
              idxs, idxd, rows, ones, zbuf, dzbuf, tails, acc_sh, deg_sh,
              sem_i, sem_g, sem_s, sem_d):
    c = lax.axis_index("c")
    s = lax.axis_index("s")

    _fill_f32_2d(zbuf, ZR, F1, 0.0)
    _fill_f32_1d(dzbuf, DEGP // NS, 0.0)
    _fill_f32_1d(ones, K, 1.0)
    for k in range(W // ZR):
        pltpu.sync_copy(zbuf, acc_sh.at[pl.ds(s * W + k * ZR, ZR)])

    @pl.when(s == NS - 1)
    def _zero_tail():
        pltpu.sync_copy(zbuf.at[pl.ds(0, N - W * NS)],
                        acc_sh.at[pl.ds(W * NS, N - W * NS)])

    pltpu.sync_copy(dzbuf, deg_sh.at[pl.ds(s * (DEGP // NS), DEGP // NS)])
    plsc.subcore_barrier()

    per_tile = E // (NC * NS)
    base = c * (E // NC) + s * per_tile
    nchunks = per_tile // K

    _agg_pipeline(x_hbm, edges_hbm, edges_hbm, acc_sh, idxs, idxd, rows,
                  sem_i, sem_g, sem_s, sem_d, base, nchunks,
                  deg_sh=deg_sh, ones=ones)
    _agg_tail(x_hbm, edges_hbm, acc_sh, tails[0], tails[1], rows, sem_g,
              base + nchunks * K, per_tile - nchunks * K,
              deg_sh=deg_sh, ones=ones)
    plsc.subcore_barrier()

    pltpu.sync_copy(acc_sh.at[pl.ds(s * W, W)],
                    out_hbm.at[c, pl.ds(s * W, W)])

    @pl.when(s == NS - 1)
    def _writeback_tail():
        pltpu.sync_copy(acc_sh.at[pl.ds(W * NS, N - W * NS)],
                        out_hbm.at[c, pl.ds(W * NS, N - W * NS)])

    DW = DEGP // NS
    pltpu.sync_copy(deg_sh.at[pl.ds(s * DW, DW)], dzbuf)
    pltpu.sync_copy(dzbuf, deg_hbm.at[pl.ds(c * DEGP + s * DW, DW)])


@functools.partial(
    pl.kernel,
    out_type=jax.ShapeDtypeStruct((NC, N, FH), jnp.float32),
    mesh=_MESH,
    scratch_types=[
        [pltpu.VMEM((K,), jnp.int32)] * 4,
        [pltpu.VMEM((K,), jnp.int32)] * 4,
        [pltpu.VMEM((K, FH), jnp.float32)] * 2,
        pltpu.VMEM((ZR, FH), jnp.float32),
        [pltpu.VMEM((32,), jnp.int32)] * 2,
        pltpu.VMEM_SHARED((N, FH), jnp.float32),
        [pltpu.SemaphoreType.DMA] * 4,
        [pltpu.SemaphoreType.DMA] * 2,
        [pltpu.SemaphoreType.DMA] * 2,
    ],
)
def _sc_conv2(h_hbm, edges_hbm, out_hbm,
              idxs, idxd, rows, zbuf, tails, acc_sh, sem_i, sem_g, sem_s):
    c = lax.axis_index("c")
    s = lax.axis_index("s")

    _fill_f32_2d(zbuf, ZR, FH, 0.0)
    for k in range(W // ZR):
        pltpu.sync_copy(zbuf, acc_sh.at[pl.ds(s * W + k * ZR, ZR)])

    @pl.when(s == NS - 1)
    def _zero_tail():
        pltpu.sync_copy(zbuf.at[pl.ds(0, N - W * NS)],
                        acc_sh.at[pl.ds(W * NS, N - W * NS)])

    plsc.subcore_barrier()

    per_tile = E // NS
    base = s * per_tile
    nchunks = per_tile // K
    row_off = c * N

    _agg_pipeline(h_hbm, edges_hbm, edges_hbm, acc_sh, idxs, idxd, rows,
                  sem_i, sem_g, sem_s, None, base, nchunks, row_off=row_off)
    _agg_tail(h_hbm, edges_hbm, acc_sh, tails[0], tails[1], rows, sem_g,
              base + nchunks * K, per_tile - nchunks * K, row_off=row_off)
    plsc.subcore_barrier()

    pltpu.sync_copy(acc_sh.at[pl.ds(s * W, W)],
                    out_hbm.at[c, pl.ds(s * W, W)])

    @pl.when(s == NS - 1)
    def _writeback_tail():
        pltpu.sync_copy(acc_sh.at[pl.ds(W * NS, N - W * NS)],
                        out_hbm.at[c, pl.ds(W * NS, N - W * NS)])


_R = 2048
_DR = DEGP // 128


def _bdot(a, b):
    return jnp.dot(a.astype(jnp.bfloat16), b.astype(jnp.bfloat16),
                   preferred_element_type=jnp.float32)


def _tc1_body(x_ref, agg_ref, deg_ref, wl_ref, bl_ref, wr_ref,
              h1_ref, degm_ref):
    a = agg_ref[0] + agg_ref[1]
    dm = jnp.maximum(deg_ref[0] + deg_ref[1], 1.0)
    inv = (1.0 / dm)[:, :, None]
    mean = (a.reshape(_R // 128, 128, F1) * inv).reshape(_R, F1)
    x1 = (_bdot(mean, wl_ref[...])
          + bl_ref[...][None, :]
          + _bdot(x_ref[...], wr_ref[...]))
    h1 = jnp.maximum(x1, 0.0)
    h1_ref[0] = h1[:, :FH]
    h1_ref[1] = h1[:, FH:]
    degm_ref[...] = dm


def _tc1(x, aggpair, degpair, wl, bl, wr):
    grid = (pl.cdiv(N, _R),)
    return pl.pallas_call(
        _tc1_body,
        grid=grid,
        in_specs=[
            pl.BlockSpec((_R, F1), lambda i: (i, 0)),
            pl.BlockSpec((NC, _R, F1), lambda i: (0, i, 0)),
            pl.BlockSpec((NC, _R // 128, 128), lambda i: (0, i, 0)),
            pl.BlockSpec((F1, 2 * FH), lambda i: (0, 0)),
            pl.BlockSpec((2 * FH,), lambda i: (0,)),
            pl.BlockSpec((F1, 2 * FH), lambda i: (0, 0)),
        ],
        out_specs=[
            pl.BlockSpec((NC, _R, FH), lambda i: (0, i, 0)),
            pl.BlockSpec((_R // 128, 128), lambda i: (i, 0)),
        ],
        out_shape=[
            jax.ShapeDtypeStruct((NC, N, FH), jnp.float32),
            jax.ShapeDtypeStruct((_DR, 128), jnp.float32),
        ],
    )(x, aggpair, degpair, wl, bl, wr)


def _tc2_body(x_ref, h1_ref, agg2_ref, degm_ref, wl2_ref, bl2_ref, wr2_ref,
              w1_ref, b1_ref, g_ref, b_ref, w2_ref, b2_ref, out_ref):
    inv = (1.0 / degm_ref[...])[:, :, None]
    h1 = jnp.concatenate([h1_ref[0], h1_ref[1]], axis=1)
    a2 = jnp.concatenate([agg2_ref[0], agg2_ref[1]], axis=1)
    m2 = (a2.reshape(_R // 128, 128, 2 * FH) * inv).reshape(_R, 2 * FH)
    x2 = (_bdot(m2, wl2_ref[...])
          + bl2_ref[...][None, :]
          + _bdot(h1, wr2_ref[...]))
    h2 = jnp.maximum(x2, 0.0)
    h = jnp.concatenate([x_ref[...], h1, h2], axis=1)
    m1 = _bdot(h, w1_ref[...])
    m1 = jnp.maximum(m1 + b1_ref[...][None, :], 0.0)
    mu = jnp.mean(m1, axis=1, keepdims=True)
    var = jnp.mean((m1 - mu) * (m1 - mu), axis=1, keepdims=True)
    ln = (m1 - mu) * lax.rsqrt(var + 1e-5) * g_ref[...][None, :] + b_ref[...][None, :]
    out_ref[...] = _bdot(ln, w2_ref[...]) + b2_ref[...][None, :]


def _tc2(x, h1pair, agg2pair, degm, wl2, bl2, wr2, w1, b1, g, b, w2, b2):
    grid = (pl.cdiv(N, _R),)
    return pl.pallas_call(
        _tc2_body,
        grid=grid,
        in_specs=[
            pl.BlockSpec((_R, F1), lambda i: (i, 0)),
            pl.BlockSpec((NC, _R, FH), lambda i: (0, i, 0)),
            pl.BlockSpec((NC, _R, FH), lambda i: (0, i, 0)),
            pl.BlockSpec((_R // 128, 128), lambda i: (i, 0)),
            pl.BlockSpec((2 * FH, 2 * FH), lambda i: (0, 0)),
            pl.BlockSpec((2 * FH,), lambda i: (0,)),
            pl.BlockSpec((2 * FH, 2 * FH), lambda i: (0, 0)),
            pl.BlockSpec((F1 + 4 * FH, 2 * FH), lambda i: (0, 0)),
            pl.BlockSpec((2 * FH,), lambda i: (0,)),
            pl.BlockSpec((2 * FH,), lambda i: (0,)),
            pl.BlockSpec((2 * FH,), lambda i: (0,)),
            pl.BlockSpec((2 * FH, 2), lambda i: (0, 0)),
            pl.BlockSpec((2,), lambda i: (0,)),
        ],
        out_specs=pl.BlockSpec((_R, 2), lambda i: (i, 0)),
        out_shape=jax.ShapeDtypeStruct((N, 2), jnp.float32),
    )(x, h1pair, agg2pair, degm, wl2, bl2, wr2, w1, b1, g, b, w2, b2)


def kernel(x, edge_index, conv1_Wl, conv1_bl, conv1_Wr, conv2_Wl, conv2_bl,
           conv2_Wr, mlp_W1, mlp_b1, ln_g, ln_b, mlp_W2, mlp_b2):
    eflat = edge_index.astype(jnp.int32).reshape(2 * E)

    aggpair, degflat = _sc_conv1(x, eflat)
    h1pair, degm = _tc1(x, aggpair, degflat.reshape(NC, _DR, 128),
                        conv1_Wl, conv1_bl, conv1_Wr)
    agg2pair = _sc_conv2(h1pair.reshape(NC * N, FH), eflat)
    out = _tc2(x, h1pair, agg2pair, degm, conv2_Wl, conv2_bl, conv2_Wr,
               mlp_W1, mlp_b1, ln_g, ln_b, mlp_W2, mlp_b2)
    return out


# speedup vs baseline: 12.3731x; 1.0040x over previous
import functools

import jax
import jax.numpy as jnp
from jax import lax
from jax.experimental import pallas as pl
from jax.experimental.pallas import tpu as pltpu
from jax.experimental.pallas import tpu_sc as plsc

N = 10000
E = 320000
F1 = 128
FH = 128
NC = 2
NS = 16
K = 128
W = 624
ZR = 48
DEGP = 10240

_MESH = plsc.VectorSubcoreMesh(
    core_axis_name="c", subcore_axis_name="s", num_cores=NC, num_subcores=NS
)


def _fill_f32_2d(ref, rows, lanes, val):
    def body(r, carry):
        for l in range(lanes // 16):
            ref[r, pl.ds(l * 16, 16)] = jnp.full((16,), val, jnp.float32)
        return carry
    lax.fori_loop(0, rows, body, 0)


def _fill_f32_1d(ref, n, val):
    def body(i, carry):
        ref[pl.ds(i * 16, 16)] = jnp.full((16,), val, jnp.float32)
        return carry
    lax.fori_loop(0, n // 16, body, 0)


def _agg_prologue(table_hbm, edges_hbm, idxs, idxd, rows, sem_i, sem_g,
                  base, row_off=None):
    for i in (0, 1):
        off = base + i * K
        pltpu.async_copy(edges_hbm.at[pl.ds(off, K)], idxs[i], sem_i[i])
        pltpu.async_copy(edges_hbm.at[pl.ds(E + off, K)], idxd[i], sem_i[i])
    pltpu.make_async_copy(edges_hbm.at[pl.ds(0, K)], idxs[0], sem_i[0]).wait()
    pltpu.make_async_copy(edges_hbm.at[pl.ds(0, K)], idxd[0], sem_i[0]).wait()
    if row_off is not None:
        def add_off(j, cy):
            sl = pl.ds(j * 16, 16)
            idxs[0][sl] = idxs[0][sl] + row_off
            return cy
        lax.fori_loop(0, K // 16, add_off, 0)
    pltpu.async_copy(table_hbm.at[idxs[0]], rows[0], sem_g[0])


def _agg_pipeline(table_hbm, src_hbm, dst_hbm, acc_sh, idxs, idxd, rows,
                  sem_i, sem_g, sem_s, sem_d, base, nchunks, row_off=None,
                  deg_sh=None, ones=None, skip_prologue=False):
    def issue_loads(i, slot):
        off = base + i * K
        pltpu.async_copy(src_hbm.at[pl.ds(off, K)], idxs[slot], sem_i[slot])
        pltpu.async_copy(dst_hbm.at[pl.ds(E + off, K)], idxd[slot],
                         sem_i[slot])

    def wait_loads(slot):
        pltpu.make_async_copy(src_hbm.at[pl.ds(0, K)], idxs[slot],
                              sem_i[slot]).wait()
        pltpu.make_async_copy(dst_hbm.at[pl.ds(0, K)], idxd[slot],
                              sem_i[slot]).wait()

    def start_gather(slot, rb):
        if row_off is not None:
            def add_off(j, cy):
                sl = pl.ds(j * 16, 16)
                idxs[slot][sl] = idxs[slot][sl] + row_off
                return cy
            lax.fori_loop(0, K // 16, add_off, 0)
        pltpu.async_copy(table_hbm.at[idxs[slot]], rows[rb], sem_g[rb])

    def wait_gather(rb):
        pltpu.make_async_copy(table_hbm.at[pl.ds(0, K)], rows[rb],
                              sem_g[rb]).wait()

    def start_scatter(slot, rb):
        pltpu.async_copy(rows[rb], acc_sh.at[idxd[slot]], sem_s[rb], add=True)
        if deg_sh is not None:
            pltpu.async_copy(ones, deg_sh.at[idxd[slot]], sem_d[rb], add=True)

    def wait_scatter(rb):
        pltpu.make_async_copy(rows[rb], acc_sh.at[pl.ds(0, K)],
                              sem_s[rb]).wait()
        if deg_sh is not None:
            pltpu.make_async_copy(ones, deg_sh.at[pl.ds(0, K)],
                                  sem_d[rb]).wait()

    if not skip_prologue:
        issue_loads(0, 0)
        issue_loads(1, 1)
        wait_loads(0)
        start_gather(0, 0)

    nquads = (nchunks + 3) // 4

    def quad(q, carry):
        for b in range(4):
            i = q * 4 + b
            s_next = (b + 1) % 4
            s_pref = (b + 2) % 4

            @pl.when(i + 1 < nchunks)
            def _gather_next():
                wait_loads(s_next)

                @pl.when(i >= 1)
                def _drain_prev_scatter():
                    wait_scatter((b + 1) % 2)

                start_gather(s_next, (b + 1) % 2)

            @pl.when(i + 2 < nchunks)
            def _prefetch_idx():
                issue_loads(i + 2, s_pref)

            @pl.when(i < nchunks)
            def _scatter():
                wait_gather(b % 2)
                start_scatter(b, b % 2)
        return carry

    lax.fori_loop(0, nquads, quad, 0)

    wait_scatter((nchunks - 2) % 2)
    wait_scatter((nchunks - 1) % 2)


def _agg_tail(table_hbm, edges_hbm, acc_sh, idxs_t, idxd_t, rows, sem_g,
              base, kt, row_off=None, deg_sh=None, ones=None):
    pltpu.sync_copy(edges_hbm.at[pl.ds(base, kt)], idxs_t)
    pltpu.sync_copy(edges_hbm.at[pl.ds(E + base, kt)], idxd_t)
    if row_off is not None:
        def add_off(j, cy):
            sl = pl.ds(j * 16, 16)
            idxs_t[sl] = idxs_t[sl] + row_off
            return cy
        lax.fori_loop(0, kt // 16, add_off, 0)
    pltpu.async_copy(table_hbm.at[idxs_t], rows[0].at[pl.ds(0, kt)],
                     sem_g[0]).wait()
    pltpu.sync_copy(rows[0].at[pl.ds(0, kt)], acc_sh.at[idxd_t], add=True)
    if deg_sh is not None:
        pltpu.sync_copy(ones.at[pl.ds(0, kt)], deg_sh.at[idxd_t], add=True)


@functools.partial(
    pl.kernel,
    out_type=[
        jax.ShapeDtypeStruct((NC, N, F1), jnp.float32),
        jax.ShapeDtypeStruct((NC * DEGP,), jnp.float32),
    ],
    mesh=_MESH,
    scratch_types=[
        [pltpu.VMEM((K,), jnp.int32)] * 4,
        [pltpu.VMEM((K,), jnp.int32)] * 4,
        [pltpu.VMEM((K, F1), jnp.float32)] * 2,
        pltpu.VMEM((K,), jnp.float32),
        pltpu.VMEM((ZR, F1), jnp.float32),
        pltpu.VMEM((DEGP // NS,), jnp.float32),
        [pltpu.VMEM((16,), jnp.int32)] * 2,
        pltpu.VMEM_SHARED((N, F1), jnp.float32),
        pltpu.VMEM_SHARED((DEGP,), jnp.float32),
        [pltpu.SemaphoreType.DMA] * 4,
        [pltpu.SemaphoreType.DMA] * 2,
        [pltpu.SemaphoreType.DMA] * 2,
        [pltpu.SemaphoreType.DMA] * 2,
    ],
)
def _sc_conv1(x_hbm, edges_hbm, out_hbm, deg_hbm,
              idxs, idxd, rows, ones, zbuf, dzbuf, tails, acc_sh, deg_sh,
              sem_i, sem_g, sem_s, sem_d):
    c = lax.axis_index("c")
    s = lax.axis_index("s")

    per_tile = E // (NC * NS)
    base = c * (E // NC) + s * per_tile
    nchunks = per_tile // K

    _agg_prologue(x_hbm, edges_hbm, idxs, idxd, rows, sem_i, sem_g, base)

    _fill_f32_2d(zbuf, ZR, F1, 0.0)
    _fill_f32_1d(dzbuf, DEGP // NS, 0.0)
    _fill_f32_1d(ones, K, 1.0)
    for k in range(W // ZR):
        pltpu.sync_copy(zbuf, acc_sh.at[pl.ds(s * W + k * ZR, ZR)])

    @pl.when(s == NS - 1)
    def _zero_tail():
        pltpu.sync_copy(zbuf.at[pl.ds(0, N - W * NS)],
                        acc_sh.at[pl.ds(W * NS, N - W * NS)])

    pltpu.sync_copy(dzbuf, deg_sh.at[pl.ds(s * (DEGP // NS), DEGP // NS)])
    plsc.subcore_barrier()

    _agg_pipeline(x_hbm, edges_hbm, edges_hbm, acc_sh, idxs, idxd, rows,
                  sem_i, sem_g, sem_s, sem_d, base, nchunks,
                  deg_sh=deg_sh, ones=ones, skip_prologue=True)
    _agg_tail(x_hbm, edges_hbm, acc_sh, tails[0], tails[1], rows, sem_g,
              base + nchunks * K, per_tile - nchunks * K,
              deg_sh=deg_sh, ones=ones)
    plsc.subcore_barrier()

    pltpu.sync_copy(acc_sh.at[pl.ds(s * W, W)],
                    out_hbm.at[c, pl.ds(s * W, W)])

    @pl.when(s == NS - 1)
    def _writeback_tail():
        pltpu.sync_copy(acc_sh.at[pl.ds(W * NS, N - W * NS)],
                        out_hbm.at[c, pl.ds(W * NS, N - W * NS)])

    DW = DEGP // NS
    pltpu.sync_copy(deg_sh.at[pl.ds(s * DW, DW)], dzbuf)
    pltpu.sync_copy(dzbuf, deg_hbm.at[pl.ds(c * DEGP + s * DW, DW)])


@functools.partial(
    pl.kernel,
    out_type=jax.ShapeDtypeStruct((NC, N, FH), jnp.float32),
    mesh=_MESH,
    scratch_types=[
        [pltpu.VMEM((K,), jnp.int32)] * 4,
        [pltpu.VMEM((K,), jnp.int32)] * 4,
        [pltpu.VMEM((K, FH), jnp.float32)] * 2,
        pltpu.VMEM((ZR, FH), jnp.float32),
        [pltpu.VMEM((32,), jnp.int32)] * 2,
        pltpu.VMEM_SHARED((N, FH), jnp.float32),
        [pltpu.SemaphoreType.DMA] * 4,
        [pltpu.SemaphoreType.DMA] * 2,
        [pltpu.SemaphoreType.DMA] * 2,
    ],
)
def _sc_conv2(h_hbm, edges_hbm, out_hbm,
              idxs, idxd, rows, zbuf, tails, acc_sh, sem_i, sem_g, sem_s):
    c = lax.axis_index("c")
    s = lax.axis_index("s")

    per_tile = E // NS
    base = s * per_tile
    nchunks = per_tile // K
    row_off = c * N

    _agg_prologue(h_hbm, edges_hbm, idxs, idxd, rows, sem_i, sem_g, base,
                  row_off=row_off)

    _fill_f32_2d(zbuf, ZR, FH, 0.0)
    for k in range(W // ZR):
        pltpu.sync_copy(zbuf, acc_sh.at[pl.ds(s * W + k * ZR, ZR)])

    @pl.when(s == NS - 1)
    def _zero_tail():
        pltpu.sync_copy(zbuf.at[pl.ds(0, N - W * NS)],
                        acc_sh.at[pl.ds(W * NS, N - W * NS)])

    plsc.subcore_barrier()

    _agg_pipeline(h_hbm, edges_hbm, edges_hbm, acc_sh, idxs, idxd, rows,
                  sem_i, sem_g, sem_s, None, base, nchunks, row_off=row_off,
                  skip_prologue=True)
    _agg_tail(h_hbm, edges_hbm, acc_sh, tails[0], tails[1], rows, sem_g,
              base + nchunks * K, per_tile - nchunks * K, row_off=row_off)
    plsc.subcore_barrier()

    pltpu.sync_copy(acc_sh.at[pl.ds(s * W, W)],
                    out_hbm.at[c, pl.ds(s * W, W)])

    @pl.when(s == NS - 1)
    def _writeback_tail():
        pltpu.sync_copy(acc_sh.at[pl.ds(W * NS, N - W * NS)],
                        out_hbm.at[c, pl.ds(W * NS, N - W * NS)])


_R = 2048
_DR = DEGP // 128


def _bdot(a, b):
    return jnp.dot(a.astype(jnp.bfloat16), b.astype(jnp.bfloat16),
                   preferred_element_type=jnp.float32)


def _tc1_body(x_ref, agg_ref, deg_ref, wl_ref, bl_ref, wr_ref,
              h1_ref, degm_ref):
    a = agg_ref[0] + agg_ref[1]
    dm = jnp.maximum(deg_ref[0] + deg_ref[1], 1.0)
    inv = (1.0 / dm)[:, :, None]
    mean = (a.reshape(_R // 128, 128, F1) * inv).reshape(_R, F1)
    x1 = (_bdot(mean, wl_ref[...])
          + bl_ref[...][None, :]
          + _bdot(x_ref[...], wr_ref[...]))
    h1 = jnp.maximum(x1, 0.0)
    h1_ref[0] = h1[:, :FH]
    h1_ref[1] = h1[:, FH:]
    degm_ref[...] = dm


def _tc1(x, aggpair, degpair, wl, bl, wr):
    grid = (pl.cdiv(N, _R),)
    return pl.pallas_call(
        _tc1_body,
        grid=grid,
        in_specs=[
            pl.BlockSpec((_R, F1), lambda i: (i, 0)),
            pl.BlockSpec((NC, _R, F1), lambda i: (0, i, 0)),
            pl.BlockSpec((NC, _R // 128, 128), lambda i: (0, i, 0)),
            pl.BlockSpec((F1, 2 * FH), lambda i: (0, 0)),
            pl.BlockSpec((2 * FH,), lambda i: (0,)),
            pl.BlockSpec((F1, 2 * FH), lambda i: (0, 0)),
        ],
        out_specs=[
            pl.BlockSpec((NC, _R, FH), lambda i: (0, i, 0)),
            pl.BlockSpec((_R // 128, 128), lambda i: (i, 0)),
        ],
        out_shape=[
            jax.ShapeDtypeStruct((NC, N, FH), jnp.float32),
            jax.ShapeDtypeStruct((_DR, 128), jnp.float32),
        ],
    )(x, aggpair, degpair, wl, bl, wr)


def _tc2_body(x_ref, h1_ref, agg2_ref, degm_ref, wl2_ref, bl2_ref, wr2_ref,
              w1_ref, b1_ref, g_ref, b_ref, w2_ref, b2_ref, out_ref):
    inv = (1.0 / degm_ref[...])[:, :, None]
    h1 = jnp.concatenate([h1_ref[0], h1_ref[1]], axis=1)
    a2 = jnp.concatenate([agg2_ref[0], agg2_ref[1]], axis=1)
    m2 = (a2.reshape(_R // 128, 128, 2 * FH) * inv).reshape(_R, 2 * FH)
    x2 = (_bdot(m2, wl2_ref[...])
          + bl2_ref[...][None, :]
          + _bdot(h1, wr2_ref[...]))
    h2 = jnp.maximum(x2, 0.0)
    h = jnp.concatenate([x_ref[...], h1, h2], axis=1)
    m1 = _bdot(h, w1_ref[...])
    m1 = jnp.maximum(m1 + b1_ref[...][None, :], 0.0)
    mu = jnp.mean(m1, axis=1, keepdims=True)
    var = jnp.mean((m1 - mu) * (m1 - mu), axis=1, keepdims=True)
    ln = (m1 - mu) * lax.rsqrt(var + 1e-5) * g_ref[...][None, :] + b_ref[...][None, :]
    out_ref[...] = _bdot(ln, w2_ref[...]) + b2_ref[...][None, :]


def _tc2(x, h1pair, agg2pair, degm, wl2, bl2, wr2, w1, b1, g, b, w2, b2):
    grid = (pl.cdiv(N, _R),)
    return pl.pallas_call(
        _tc2_body,
        grid=grid,
        in_specs=[
            pl.BlockSpec((_R, F1), lambda i: (i, 0)),
            pl.BlockSpec((NC, _R, FH), lambda i: (0, i, 0)),
            pl.BlockSpec((NC, _R, FH), lambda i: (0, i, 0)),
            pl.BlockSpec((_R // 128, 128), lambda i: (i, 0)),
            pl.BlockSpec((2 * FH, 2 * FH), lambda i: (0, 0)),
            pl.BlockSpec((2 * FH,), lambda i: (0,)),
            pl.BlockSpec((2 * FH, 2 * FH), lambda i: (0, 0)),
            pl.BlockSpec((F1 + 4 * FH, 2 * FH), lambda i: (0, 0)),
            pl.BlockSpec((2 * FH,), lambda i: (0,)),
            pl.BlockSpec((2 * FH,), lambda i: (0,)),
            pl.BlockSpec((2 * FH,), lambda i: (0,)),
            pl.BlockSpec((2 * FH, 2), lambda i: (0, 0)),
            pl.BlockSpec((2,), lambda i: (0,)),
        ],
        out_specs=pl.BlockSpec((_R, 2), lambda i: (i, 0)),
        out_shape=jax.ShapeDtypeStruct((N, 2), jnp.float32),
    )(x, h1pair, agg2pair, degm, wl2, bl2, wr2, w1, b1, g, b, w2, b2)


def kernel(x, edge_index, conv1_Wl, conv1_bl, conv1_Wr, conv2_Wl, conv2_bl,
           conv2_Wr, mlp_W1, mlp_b1, ln_g, ln_b, mlp_W2, mlp_b2):
    eflat = edge_index.astype(jnp.int32).reshape(2 * E)

    aggpair, degflat = _sc_conv1(x, eflat)
    h1pair, degm = _tc1(x, aggpair, degflat.reshape(NC, _DR, 128),
                        conv1_Wl, conv1_bl, conv1_Wr)
    agg2pair = _sc_conv2(h1pair.reshape(NC * N, FH), eflat)
    out = _tc2(x, h1pair, agg2pair, degm, conv2_Wl, conv2_bl, conv2_Wr,
               mlp_W1, mlp_b1, ln_g, ln_b, mlp_W2, mlp_b2)
    return out


# speedup vs baseline: 12.4868x; 1.0092x over previous
import functools

import jax
import jax.numpy as jnp
from jax import lax
from jax.experimental import pallas as pl
from jax.experimental.pallas import tpu as pltpu
from jax.experimental.pallas import tpu_sc as plsc

N = 10000
E = 320000
F1 = 128
FH = 128
NC = 2
NS = 16
K = 128
W = 624
ZR = 48
DEGP = 10240

_MESH = plsc.VectorSubcoreMesh(
    core_axis_name="c", subcore_axis_name="s", num_cores=NC, num_subcores=NS
)


def _fill_f32_2d(ref, rows, lanes, val):
    def body(r, carry):
        for l in range(lanes // 16):
            ref[r, pl.ds(l * 16, 16)] = jnp.full((16,), val, jnp.float32)
        return carry
    lax.fori_loop(0, rows, body, 0)


def _fill_f32_1d(ref, n, val):
    def body(i, carry):
        ref[pl.ds(i * 16, 16)] = jnp.full((16,), val, jnp.float32)
        return carry
    lax.fori_loop(0, n // 16, body, 0)


def _agg_prologue(table_hbm, edges_hbm, ebuf, rows, sem_i, sem_g,
                  cbase, row_off=None):
    for i in (0, 1):
        co = cbase + i * NS
        pltpu.async_copy(edges_hbm.at[:, pl.ds(co * K, K)], ebuf[i],
                         sem_i[i])
    pltpu.make_async_copy(edges_hbm.at[:, pl.ds(0, K)], ebuf[0],
                          sem_i[0]).wait()
    if row_off is not None:
        def add_off(j, cy):
            ebuf[0][0, pl.ds(j * 16, 16)] = (
                ebuf[0][0, pl.ds(j * 16, 16)] + row_off)
            return cy
        lax.fori_loop(0, K // 16, add_off, 0)
    pltpu.async_copy(table_hbm.at[ebuf[0].at[0]], rows[0], sem_g[0])


def _agg_pipeline(table_hbm, edges_hbm, acc_sh, ebuf, rows,
                  sem_i, sem_g, sem_s, sem_d, cbase, nchunks, nlim,
                  row_off=None, deg_sh=None, ones=None):
    def issue_loads(i, slot):
        co = cbase + i * NS
        pltpu.async_copy(edges_hbm.at[:, pl.ds(co * K, K)], ebuf[slot],
                         sem_i[slot])

    def wait_loads(slot):
        pltpu.make_async_copy(edges_hbm.at[:, pl.ds(0, K)], ebuf[slot],
                              sem_i[slot]).wait()

    def start_gather(slot, rb):
        if row_off is not None:
            def add_off(j, cy):
                ebuf[slot][0, pl.ds(j * 16, 16)] = (
                    ebuf[slot][0, pl.ds(j * 16, 16)] + row_off)
                return cy
            lax.fori_loop(0, K // 16, add_off, 0)
        pltpu.async_copy(table_hbm.at[ebuf[slot].at[0]], rows[rb], sem_g[rb])

    def wait_gather(rb):
        pltpu.make_async_copy(table_hbm.at[pl.ds(0, K)], rows[rb],
                              sem_g[rb]).wait()

    def start_scatter(slot, rb):
        pltpu.async_copy(rows[rb], acc_sh.at[ebuf[slot].at[1]], sem_s[rb],
                         add=True)
        if deg_sh is not None:
            pltpu.async_copy(ones, deg_sh.at[ebuf[slot].at[1]], sem_d[rb],
                             add=True)

    def wait_scatter(rb):
        pltpu.make_async_copy(rows[rb], acc_sh.at[pl.ds(0, K)],
                              sem_s[rb]).wait()
        if deg_sh is not None:
            pltpu.make_async_copy(ones, deg_sh.at[pl.ds(0, K)],
                                  sem_d[rb]).wait()

    nquads = (nchunks + 3) // 4

    def quad(q, carry):
        for b in range(4):
            i = q * 4 + b
            s_next = (b + 1) % 4
            s_pref = (b + 2) % 4

            @pl.when(i + 1 < nlim)
            def _gather_next():
                wait_loads(s_next)

                @pl.when(i >= 1)
                def _drain_prev_scatter():
                    wait_scatter((b + 1) % 2)

                start_gather(s_next, (b + 1) % 2)

            @pl.when(i + 2 < nlim)
            def _prefetch_idx():
                issue_loads(i + 2, s_pref)

            @pl.when(i < nlim)
            def _scatter():
                wait_gather(b % 2)
                start_scatter(b, b % 2)
        return carry

    lax.fori_loop(0, nquads, quad, 0)

    wait_scatter(0)
    wait_scatter(1)


@functools.partial(
    pl.kernel,
    out_type=[
        jax.ShapeDtypeStruct((NC, N, F1), jnp.float32),
        jax.ShapeDtypeStruct((NC * DEGP,), jnp.float32),
    ],
    mesh=_MESH,
    scratch_types=[
        [pltpu.VMEM((2, K), jnp.int32)] * 4,
        [pltpu.VMEM((K, F1), jnp.float32)] * 2,
        pltpu.VMEM((K,), jnp.float32),
        pltpu.VMEM((ZR, F1), jnp.float32),
        pltpu.VMEM((DEGP // NS,), jnp.float32),
        pltpu.VMEM_SHARED((N, F1), jnp.float32),
        pltpu.VMEM_SHARED((DEGP,), jnp.float32),
        [pltpu.SemaphoreType.DMA] * 4,
        [pltpu.SemaphoreType.DMA] * 2,
        [pltpu.SemaphoreType.DMA] * 2,
        [pltpu.SemaphoreType.DMA] * 2,
    ],
)
def _sc_conv1(x_hbm, edges_hbm, out_hbm, deg_hbm,
              ebuf, rows, ones, zbuf, dzbuf, acc_sh, deg_sh,
              sem_i, sem_g, sem_s, sem_d):
    c = lax.axis_index("c")
    s = lax.axis_index("s")

    cpc = (E // NC) // K
    cbase = c * cpc + s
    nchunks = cpc // NS + 1
    nlim = cpc // NS + (s < cpc % NS).astype(jnp.int32)

    _agg_prologue(x_hbm, edges_hbm, ebuf, rows, sem_i, sem_g, cbase)

    _fill_f32_2d(zbuf, ZR, F1, 0.0)
    _fill_f32_1d(dzbuf, DEGP // NS, 0.0)
    _fill_f32_1d(ones, K, 1.0)
    for k in range(W // ZR):
        pltpu.sync_copy(zbuf, acc_sh.at[pl.ds(s * W + k * ZR, ZR)])

    @pl.when(s == NS - 1)
    def _zero_tail():
        pltpu.sync_copy(zbuf.at[pl.ds(0, N - W * NS)],
                        acc_sh.at[pl.ds(W * NS, N - W * NS)])

    pltpu.sync_copy(dzbuf, deg_sh.at[pl.ds(s * (DEGP // NS), DEGP // NS)])
    plsc.subcore_barrier()

    _agg_pipeline(x_hbm, edges_hbm, acc_sh, ebuf, rows,
                  sem_i, sem_g, sem_s, sem_d, cbase, nchunks, nlim,
                  deg_sh=deg_sh, ones=ones)
    plsc.subcore_barrier()

    pltpu.sync_copy(acc_sh.at[pl.ds(s * W, W)],
                    out_hbm.at[c, pl.ds(s * W, W)])

    @pl.when(s == NS - 1)
    def _writeback_tail():
        pltpu.sync_copy(acc_sh.at[pl.ds(W * NS, N - W * NS)],
                        out_hbm.at[c, pl.ds(W * NS, N - W * NS)])

    DW = DEGP // NS
    pltpu.sync_copy(deg_sh.at[pl.ds(s * DW, DW)], dzbuf)
    pltpu.sync_copy(dzbuf, deg_hbm.at[pl.ds(c * DEGP + s * DW, DW)])


@functools.partial(
    pl.kernel,
    out_type=jax.ShapeDtypeStruct((NC, N, FH), jnp.float32),
    mesh=_MESH,
    scratch_types=[
        [pltpu.VMEM((2, K), jnp.int32)] * 4,
        [pltpu.VMEM((K, FH), jnp.float32)] * 2,
        pltpu.VMEM((ZR, FH), jnp.float32),
        pltpu.VMEM_SHARED((N, FH), jnp.float32),
        [pltpu.SemaphoreType.DMA] * 4,
        [pltpu.SemaphoreType.DMA] * 2,
        [pltpu.SemaphoreType.DMA] * 2,
    ],
)
def _sc_conv2(h_hbm, edges_hbm, out_hbm,
              ebuf, rows, zbuf, acc_sh, sem_i, sem_g, sem_s):
    c = lax.axis_index("c")
    s = lax.axis_index("s")

    cpc = E // K
    cbase = s
    nchunks = cpc // NS + 1
    nlim = cpc // NS + (s < cpc % NS).astype(jnp.int32)
    row_off = c * N

    _agg_prologue(h_hbm, edges_hbm, ebuf, rows, sem_i, sem_g, cbase,
                  row_off=row_off)

    _fill_f32_2d(zbuf, ZR, FH, 0.0)
    for k in range(W // ZR):
        pltpu.sync_copy(zbuf, acc_sh.at[pl.ds(s * W + k * ZR, ZR)])

    @pl.when(s == NS - 1)
    def _zero_tail():
        pltpu.sync_copy(zbuf.at[pl.ds(0, N - W * NS)],
                        acc_sh.at[pl.ds(W * NS, N - W * NS)])

    plsc.subcore_barrier()

    _agg_pipeline(h_hbm, edges_hbm, acc_sh, ebuf, rows,
                  sem_i, sem_g, sem_s, None, cbase, nchunks, nlim,
                  row_off=row_off)
    plsc.subcore_barrier()

    pltpu.sync_copy(acc_sh.at[pl.ds(s * W, W)],
                    out_hbm.at[c, pl.ds(s * W, W)])

    @pl.when(s == NS - 1)
    def _writeback_tail():
        pltpu.sync_copy(acc_sh.at[pl.ds(W * NS, N - W * NS)],
                        out_hbm.at[c, pl.ds(W * NS, N - W * NS)])


_R = 2048
_DR = DEGP // 128


def _bdot(a, b):
    return jnp.dot(a.astype(jnp.bfloat16), b.astype(jnp.bfloat16),
                   preferred_element_type=jnp.float32)


def _tc1_body(x_ref, agg_ref, deg_ref, wl_ref, bl_ref, wr_ref,
              h1_ref, degm_ref):
    a = agg_ref[0] + agg_ref[1]
    dm = jnp.maximum(deg_ref[0] + deg_ref[1], 1.0)
    inv = (1.0 / dm)[:, :, None]
    mean = (a.reshape(_R // 128, 128, F1) * inv).reshape(_R, F1)
    x1 = (_bdot(mean, wl_ref[...])
          + bl_ref[...][None, :]
          + _bdot(x_ref[...], wr_ref[...]))
    h1 = jnp.maximum(x1, 0.0)
    h1_ref[0] = h1[:, :FH]
    h1_ref[1] = h1[:, FH:]
    degm_ref[...] = dm


def _tc1(x, aggpair, degpair, wl, bl, wr):
    grid = (pl.cdiv(N, _R),)
    return pl.pallas_call(
        _tc1_body,
        grid=grid,
        in_specs=[
            pl.BlockSpec((_R, F1), lambda i: (i, 0)),
            pl.BlockSpec((NC, _R, F1), lambda i: (0, i, 0)),
            pl.BlockSpec((NC, _R // 128, 128), lambda i: (0, i, 0)),
            pl.BlockSpec((F1, 2 * FH), lambda i: (0, 0)),
            pl.BlockSpec((2 * FH,), lambda i: (0,)),
            pl.BlockSpec((F1, 2 * FH), lambda i: (0, 0)),
        ],
        out_specs=[
            pl.BlockSpec((NC, _R, FH), lambda i: (0, i, 0)),
            pl.BlockSpec((_R // 128, 128), lambda i: (i, 0)),
        ],
        out_shape=[
            jax.ShapeDtypeStruct((NC, N, FH), jnp.float32),
            jax.ShapeDtypeStruct((_DR, 128), jnp.float32),
        ],
    )(x, aggpair, degpair, wl, bl, wr)


def _tc2_body(x_ref, h1_ref, agg2_ref, degm_ref, wl2_ref, bl2_ref, wr2_ref,
              w1_ref, b1_ref, g_ref, b_ref, w2_ref, b2_ref, out_ref):
    inv = (1.0 / degm_ref[...])[:, :, None]
    h1 = jnp.concatenate([h1_ref[0], h1_ref[1]], axis=1)
    a2 = jnp.concatenate([agg2_ref[0], agg2_ref[1]], axis=1)
    m2 = (a2.reshape(_R // 128, 128, 2 * FH) * inv).reshape(_R, 2 * FH)
    x2 = (_bdot(m2, wl2_ref[...])
          + bl2_ref[...][None, :]
          + _bdot(h1, wr2_ref[...]))
    h2 = jnp.maximum(x2, 0.0)
    h = jnp.concatenate([x_ref[...], h1, h2], axis=1)
    m1 = _bdot(h, w1_ref[...])
    m1 = jnp.maximum(m1 + b1_ref[...][None, :], 0.0)
    mu = jnp.mean(m1, axis=1, keepdims=True)
    var = jnp.mean((m1 - mu) * (m1 - mu), axis=1, keepdims=True)
    ln = (m1 - mu) * lax.rsqrt(var + 1e-5) * g_ref[...][None, :] + b_ref[...][None, :]
    out_ref[...] = _bdot(ln, w2_ref[...]) + b2_ref[...][None, :]


def _tc2(x, h1pair, agg2pair, degm, wl2, bl2, wr2, w1, b1, g, b, w2, b2):
    grid = (pl.cdiv(N, _R),)
    return pl.pallas_call(
        _tc2_body,
        grid=grid,
        in_specs=[
            pl.BlockSpec((_R, F1), lambda i: (i, 0)),
            pl.BlockSpec((NC, _R, FH), lambda i: (0, i, 0)),
            pl.BlockSpec((NC, _R, FH), lambda i: (0, i, 0)),
            pl.BlockSpec((_R // 128, 128), lambda i: (i, 0)),
            pl.BlockSpec((2 * FH, 2 * FH), lambda i: (0, 0)),
            pl.BlockSpec((2 * FH,), lambda i: (0,)),
            pl.BlockSpec((2 * FH, 2 * FH), lambda i: (0, 0)),
            pl.BlockSpec((F1 + 4 * FH, 2 * FH), lambda i: (0, 0)),
            pl.BlockSpec((2 * FH,), lambda i: (0,)),
            pl.BlockSpec((2 * FH,), lambda i: (0,)),
            pl.BlockSpec((2 * FH,), lambda i: (0,)),
            pl.BlockSpec((2 * FH, 2), lambda i: (0, 0)),
            pl.BlockSpec((2,), lambda i: (0,)),
        ],
        out_specs=pl.BlockSpec((_R, 2), lambda i: (i, 0)),
        out_shape=jax.ShapeDtypeStruct((N, 2), jnp.float32),
    )(x, h1pair, agg2pair, degm, wl2, bl2, wr2, w1, b1, g, b, w2, b2)


def kernel(x, edge_index, conv1_Wl, conv1_bl, conv1_Wr, conv2_Wl, conv2_bl,
           conv2_Wr, mlp_W1, mlp_b1, ln_g, ln_b, mlp_W2, mlp_b2):
    ei = edge_index.astype(jnp.int32)

    aggpair, degflat = _sc_conv1(x, ei)
    h1pair, degm = _tc1(x, aggpair, degflat.reshape(NC, _DR, 128),
                        conv1_Wl, conv1_bl, conv1_Wr)
    agg2pair = _sc_conv2(h1pair.reshape(NC * N, FH), ei)
    out = _tc2(x, h1pair, agg2pair, degm, conv2_Wl, conv2_bl, conv2_Wr,
               mlp_W1, mlp_b1, ln_g, ln_b, mlp_W2, mlp_b2)
    return out
